# baseline (speedup 1.0000x reference)
import functools

import jax
import jax.numpy as jnp
from jax import lax
from jax.experimental import pallas as pl
from jax.experimental.pallas import tpu as pltpu
from jax.experimental.pallas import tpu_sc as plsc

N = 10000
M = 5000
NNZ = 320000
DIN = 128
DOUT = 64
DS = 10
NEG_SLOPE = 0.2

NC = 2
NS = 16
NW = NC * NS
LANES = 16

NP = 10240
MP = 5120
NPS = NP // NS
MPS = MP // NS

CHUNK = 128
PC = 80
PT = PC * CHUNK
NNZP = NW * PT

BLK = 256
NBN = NP // BLK
NBM = MP // BLK


def _elu(x):
    return jnp.where(x > 0, x, jnp.exp(jnp.minimum(x, 0.0)) - 1.0)


def _tc1a_body(x_ref, wx_ref, wv_ref, wa_ref, bx_ref, bv_ref,
               xi_ref, xf_ref, sv_ref, gmax_ref):
    b = pl.program_id(0)
    x = x_ref[...]
    xf = jnp.dot(x, wv_ref[...], preferred_element_type=jnp.float32) + bv_ref[...]
    xi_ref[...] = jnp.dot(x, wx_ref[...], preferred_element_type=jnp.float32) + bx_ref[...]
    xf_ref[...] = xf
    sv = jnp.dot(xf, wa_ref[...], preferred_element_type=jnp.float32)
    sv = jnp.where(sv > 0, sv, NEG_SLOPE * sv)
    sv_ref[...] = sv
    m2 = jnp.max(sv, axis=(0, 1), keepdims=True)

    @pl.when(b == 0)
    def _():
        gmax_ref[...] = m2

    @pl.when(b > 0)
    def _():
        gmax_ref[...] = jnp.maximum(gmax_ref[...], m2)


def _tc1a(x_pad, w_x, w_vertex, w_att, bx2, bv2):
    return pl.pallas_call(
        _tc1a_body,
        grid=(NBN,),
        in_specs=[
            pl.BlockSpec((BLK, DIN), lambda b: (b, 0)),
            pl.BlockSpec((DIN, DOUT), lambda b: (0, 0)),
            pl.BlockSpec((DIN, DOUT), lambda b: (0, 0)),
            pl.BlockSpec((DOUT, 1), lambda b: (0, 0)),
            pl.BlockSpec((1, DOUT), lambda b: (0, 0)),
            pl.BlockSpec((1, DOUT), lambda b: (0, 0)),
        ],
        out_specs=[
            pl.BlockSpec((BLK, DOUT), lambda b: (b, 0)),
            pl.BlockSpec((BLK, DOUT), lambda b: (b, 0)),
            pl.BlockSpec((BLK, 1), lambda b: (b, 0)),
            pl.BlockSpec((1, 1), lambda b: (0, 0)),
        ],
        out_shape=[
            jax.ShapeDtypeStruct((NP, DOUT), jnp.float32),
            jax.ShapeDtypeStruct((NP, DOUT), jnp.float32),
            jax.ShapeDtypeStruct((NP, 1), jnp.float32),
            jax.ShapeDtypeStruct((1, 1), jnp.float32),
        ],
    )(x_pad, w_x, w_vertex, w_att, bx2, bv2)


def _tc1b_body(xf_ref, sv_ref, gmax_ref, g_ref, esv_ref):
    esv = jnp.exp(sv_ref[...] - gmax_ref[...])
    g_ref[...] = xf_ref[...] * esv
    esv_ref[...] = esv


def _tc1b(xf, sv, gmax):
    return pl.pallas_call(
        _tc1b_body,
        grid=(NBN,),
        in_specs=[
            pl.BlockSpec((BLK, DOUT), lambda b: (b, 0)),
            pl.BlockSpec((BLK, 1), lambda b: (b, 0)),
            pl.BlockSpec((1, 1), lambda b: (0, 0)),
        ],
        out_specs=[
            pl.BlockSpec((BLK, DOUT), lambda b: (b, 0)),
            pl.BlockSpec((BLK, 1), lambda b: (b, 0)),
        ],
        out_shape=[
            jax.ShapeDtypeStruct((NP, DOUT), jnp.float32),
            jax.ShapeDtypeStruct((NP, 1), jnp.float32),
        ],
    )(xf, sv, gmax)


def _tc2_body(np_ref, dp_ref, s2_ref, w1_ref, w2_ref, be_ref, y_ref):
    num = np_ref[0] + np_ref[1]
    den = jnp.maximum(dp_ref[0] + dp_ref[1], 1e-12)
    yv = _elu(num / den[:, None])
    y_ref[...] = (
        jnp.dot(yv, w1_ref[...], preferred_element_type=jnp.float32)
        + jnp.dot(s2_ref[...], w2_ref[...], preferred_element_type=jnp.float32)
        + be_ref[...]
    )


def _tc2(num_p, den_p, s2, w1, w2, be2):
    return pl.pallas_call(
        _tc2_body,
        grid=(NBM,),
        in_specs=[
            pl.BlockSpec((NC, BLK, DOUT), lambda b: (0, b, 0)),
            pl.BlockSpec((NC, BLK), lambda b: (0, b)),
            pl.BlockSpec((BLK, DOUT), lambda b: (b, 0)),
            pl.BlockSpec((DOUT, DOUT), lambda b: (0, 0)),
            pl.BlockSpec((DOUT, DOUT), lambda b: (0, 0)),
            pl.BlockSpec((1, DOUT), lambda b: (0, 0)),
        ],
        out_specs=pl.BlockSpec((BLK, DOUT), lambda b: (b, 0)),
        out_shape=jax.ShapeDtypeStruct((MP, DOUT), jnp.float32),
    )(num_p, den_p, s2, w1, w2, be2)


def _tc3_body(xp_ref, cp_ref, xi_ref, out_ref):
    xs = xp_ref[0] + xp_ref[1]
    cnt = jnp.maximum(cp_ref[0] + cp_ref[1], 1.0)
    out_ref[...] = _elu(xs / cnt[:, None]) + xi_ref[...]


def _tc3(xs_p, cnt_p, x_init):
    return pl.pallas_call(
        _tc3_body,
        grid=(NBN,),
        in_specs=[
            pl.BlockSpec((NC, BLK, DOUT), lambda b: (0, b, 0)),
            pl.BlockSpec((NC, BLK), lambda b: (0, b)),
            pl.BlockSpec((BLK, DOUT), lambda b: (b, 0)),
        ],
        out_specs=pl.BlockSpec((BLK, DOUT), lambda b: (b, 0)),
        out_shape=jax.ShapeDtypeStruct((NP, DOUT), jnp.float32),
    )(xs_p, cnt_p, x_init)


_MESH = plsc.VectorSubcoreMesh(core_axis_name="c", subcore_axis_name="s")

_Z16 = functools.partial(jnp.zeros, (LANES,), jnp.float32)


def _zero_1d(ref, n):
    def body(i, _):
        ref[pl.ds(i * LANES, LANES)] = _Z16()
        return 0
    lax.fori_loop(0, n // LANES, body, 0)


def _zero_rows(ref, rows):
    def body(i, _):
        for k in range(DOUT // LANES):
            ref[i, pl.ds(k * LANES, LANES)] = _Z16()
        return 0
    lax.fori_loop(0, rows, body, 0)


@functools.partial(
    pl.kernel,
    out_type=[
        jax.ShapeDtypeStruct((NC, MP, DOUT), jnp.float32),
        jax.ShapeDtypeStruct((NC, MP), jnp.float32),
        jax.ShapeDtypeStruct((NC, NP), jnp.float32),
    ],
    mesh=_MESH,
    scratch_types=[
        pltpu.VMEM((PC, CHUNK), jnp.int32),
        pltpu.VMEM((PC, CHUNK), jnp.int32),
        pltpu.VMEM((NP,), jnp.float32),
        pltpu.VMEM((2, CHUNK, DOUT), jnp.float32),
        pltpu.VMEM((MP,), jnp.float32),
        pltpu.VMEM((NP,), jnp.float32),
        pltpu.VMEM((NS, MPS), jnp.float32),
        pltpu.VMEM((NS, NPS), jnp.float32),
        pltpu.VMEM((MPS,), jnp.float32),
        pltpu.VMEM((NPS,), jnp.float32),
        pltpu.VMEM_SHARED((MP, DOUT), jnp.float32),
        pltpu.VMEM_SHARED((NS, MP), jnp.float32),
        pltpu.VMEM_SHARED((NS, NP), jnp.float32),
        pltpu.SemaphoreType.DMA,
        pltpu.SemaphoreType.DMA,
    ],
    compiler_params=pltpu.CompilerParams(use_tc_tiling_on_sc=False, needs_layout_passes=False),
    name="sc1_v2e",
)
def _sc1(g_hbm, esv_hbm, src_hbm, dst_hbm, num_out, den_out, cnt_out,
         src_v, dst_v, esv_v, rowbuf, den_loc, cnt_loc,
         denred, cntred, denacc, cntacc, num_sh, den_sh, cnt_sh, sem0, sem1):
    cid = lax.axis_index("c")
    sid = lax.axis_index("s")
    wid = cid * NS + sid

    pltpu.sync_copy(src_hbm.at[wid], src_v)
    pltpu.sync_copy(dst_hbm.at[wid], dst_v)
    pltpu.sync_copy(esv_hbm, esv_v)

    _zero_rows(rowbuf.at[0], CHUNK)
    _zero_rows(rowbuf.at[1], CHUNK)
    _zero_1d(den_loc, MP)
    _zero_1d(cnt_loc, NP)
    pltpu.sync_copy(rowbuf.at[0], num_sh.at[pl.ds(sid * MPS, CHUNK)])
    pltpu.sync_copy(rowbuf.at[1], num_sh.at[pl.ds(sid * MPS + CHUNK, CHUNK)])
    pltpu.sync_copy(rowbuf.at[0, pl.ds(0, MPS - 2 * CHUNK)],
                    num_sh.at[pl.ds(sid * MPS + 2 * CHUNK, MPS - 2 * CHUNK)])
    plsc.subcore_barrier()

    pltpu.async_copy(g_hbm.at[src_v.at[0]], rowbuf.at[0], sem0)

    ones16 = jnp.ones((LANES,), jnp.float32)
    sems = (sem0, sem1)

    def chunk_work(jj, b):
        sem = sems[b]
        osem = sems[1 - b]
        pltpu.make_async_copy(g_hbm.at[src_v.at[jj]], rowbuf.at[b], sem).wait()

        @pl.when(jj + 1 < PC)
        def _():
            pltpu.async_copy(g_hbm.at[src_v.at[jj + 1]], rowbuf.at[1 - b], osem)

        desc = pltpu.async_copy(rowbuf.at[b], num_sh.at[dst_v.at[jj]], sem,
                                add=True)
        for k in range(CHUNK // LANES):
            sidx = src_v[jj, pl.ds(k * LANES, LANES)]
            didx = dst_v[jj, pl.ds(k * LANES, LANES)]
            e = plsc.load_gather(esv_v, [sidx])
            plsc.addupdate_scatter(den_loc, [didx], e)
            plsc.addupdate_scatter(cnt_loc, [sidx], ones16)
        desc.wait()

    def body(jh, _):
        chunk_work(2 * jh, 0)
        chunk_work(2 * jh + 1, 1)
        return 0

    lax.fori_loop(0, PC // 2, body, 0)

    plsc.subcore_barrier()
    pltpu.sync_copy(den_loc, den_sh.at[sid])
    pltpu.sync_copy(cnt_loc, cnt_sh.at[sid])
    plsc.subcore_barrier()

    pltpu.sync_copy(num_sh.at[pl.ds(sid * MPS, MPS)],
                    num_out.at[cid, pl.ds(sid * MPS, MPS)])

    for r in range(NS):
        pltpu.sync_copy(den_sh.at[r, pl.ds(sid * MPS, MPS)], denred.at[r])

    def dred(i, _):
        acc = _Z16()
        for r in range(NS):
            acc = acc + denred[r, pl.ds(i * LANES, LANES)]
        denacc[pl.ds(i * LANES, LANES)] = acc
        return 0

    lax.fori_loop(0, MPS // LANES, dred, 0)
    pltpu.sync_copy(denacc, den_out.at[cid, pl.ds(sid * MPS, MPS)])

    for r in range(NS):
        pltpu.sync_copy(cnt_sh.at[r, pl.ds(sid * NPS, NPS)], cntred.at[r])

    def cred(i, _):
        acc = _Z16()
        for r in range(NS):
            acc = acc + cntred[r, pl.ds(i * LANES, LANES)]
        cntacc[pl.ds(i * LANES, LANES)] = acc
        return 0

    lax.fori_loop(0, NPS // LANES, cred, 0)
    pltpu.sync_copy(cntacc, cnt_out.at[cid, pl.ds(sid * NPS, NPS)])


@functools.partial(
    pl.kernel,
    out_type=jax.ShapeDtypeStruct((NC, NP, DOUT), jnp.float32),
    mesh=_MESH,
    scratch_types=[
        pltpu.VMEM((PC, CHUNK), jnp.int32),
        pltpu.VMEM((PC, CHUNK), jnp.int32),
        pltpu.VMEM((2, CHUNK, DOUT), jnp.float32),
        pltpu.VMEM((NPS, DOUT), jnp.float32),
        pltpu.VMEM_SHARED((NP, DOUT), jnp.float32),
        pltpu.SemaphoreType.DMA,
        pltpu.SemaphoreType.DMA,
    ],
    compiler_params=pltpu.CompilerParams(use_tc_tiling_on_sc=False, needs_layout_passes=False),
    name="sc2_e2v",
)
def _sc2(y_hbm, src_hbm, dst_hbm, xs_out,
         src_v, dst_v, rowbuf, zrows, xacc, sem0, sem1):
    cid = lax.axis_index("c")
    sid = lax.axis_index("s")
    wid = cid * NS + sid

    pltpu.sync_copy(src_hbm.at[wid], src_v)
    pltpu.sync_copy(dst_hbm.at[wid], dst_v)

    _zero_rows(zrows, NPS)
    pltpu.sync_copy(zrows, xacc.at[pl.ds(sid * NPS, NPS)])
    plsc.subcore_barrier()

    pltpu.async_copy(y_hbm.at[dst_v.at[0]], rowbuf.at[0], sem0)
    sems = (sem0, sem1)

    def chunk_work(jj, b):
        sem = sems[b]
        osem = sems[1 - b]
        pltpu.make_async_copy(y_hbm.at[dst_v.at[jj]], rowbuf.at[b], sem).wait()

        @pl.when(jj + 1 < PC)
        def _():
            pltpu.async_copy(y_hbm.at[dst_v.at[jj + 1]], rowbuf.at[1 - b], osem)

        pltpu.async_copy(rowbuf.at[b], xacc.at[src_v.at[jj]], sem,
                         add=True).wait()

    def body(jh, _):
        chunk_work(2 * jh, 0)
        chunk_work(2 * jh + 1, 1)
        return 0

    lax.fori_loop(0, PC // 2, body, 0)

    plsc.subcore_barrier()
    pltpu.sync_copy(xacc.at[pl.ds(sid * NPS, NPS)],
                    xs_out.at[cid, pl.ds(sid * NPS, NPS)])


def kernel(X, v2e_src, v2e_dst, S_features, W_x, b_x, W_vertex, b_vertex,
           W_group, b_group, W_att, W_e2v, b_e2v):
    x_pad = jnp.pad(X, ((0, NP - N), (0, 0)))
    npad = NNZP - NNZ
    pad_src = (N + jnp.arange(npad, dtype=jnp.int32) % (NP - N)).astype(jnp.int32)
    pad_dst = (M + jnp.arange(npad, dtype=jnp.int32) % (MP - M)).astype(jnp.int32)
    src_all = jnp.concatenate([v2e_src, pad_src])
    dst_all = jnp.concatenate([v2e_dst, pad_dst])
    src_s = jnp.swapaxes(src_all.reshape(NW, CHUNK, PC), 1, 2)
    dst_s = jnp.swapaxes(dst_all.reshape(NW, CHUNK, PC), 1, 2)

    s2 = jnp.pad(S_features, ((0, MP - M), (0, DOUT - DS)))
    w1 = W_e2v[:DOUT]
    w2 = jnp.pad(W_e2v[DOUT:], ((0, DOUT - DS), (0, 0)))
    bx2 = b_x[None, :]
    bv2 = b_vertex[None, :]
    be2 = b_e2v[None, :]

    x_init, xf, sv, gmax = _tc1a(x_pad, W_x, W_vertex, W_att, bx2, bv2)
    g, esv2 = _tc1b(xf, sv, gmax)
    esv = esv2.reshape(NP)

    num_p, den_p, cnt_p = _sc1(g, esv, src_s, dst_s)
    y = _tc2(num_p, den_p, s2, w1, w2, be2)
    xs_p = _sc2(y, src_s, dst_s)
    out = _tc3(xs_p, cnt_p, x_init)
    return out[:N]


# speedup vs baseline: 1.1426x; 1.1426x over previous
import functools

import jax
import jax.numpy as jnp
from jax import lax
from jax.experimental import pallas as pl
from jax.experimental.pallas import tpu as pltpu
from jax.experimental.pallas import tpu_sc as plsc

N = 10000
M = 5000
NNZ = 320000
DIN = 128
DOUT = 64
DS = 10
NEG_SLOPE = 0.2

NC = 2
NS = 16
NW = NC * NS
LANES = 16

NP = 10240
MP = 5120
NPS = NP // NS
MPS = MP // NS

CHUNK = 128
PC = 80
PT = PC * CHUNK
NNZP = NW * PT

BLK = 256
NBN = NP // BLK
NBM = MP // BLK


def _elu(x):
    return jnp.where(x > 0, x, jnp.exp(jnp.minimum(x, 0.0)) - 1.0)


def _tc1_body(x_ref, wx_ref, wv_ref, wa_ref, bx_ref, bv_ref,
              xi_ref, g_ref, esv_ref, xi_s, xf_s, sv_s, gmax_s):
    p = pl.program_id(0)
    b = pl.program_id(1)

    @pl.when(p == 0)
    def _():
        x = x_ref[...]
        xf = jnp.dot(x, wv_ref[...], preferred_element_type=jnp.float32) + bv_ref[...]
        xi = jnp.dot(x, wx_ref[...], preferred_element_type=jnp.float32) + bx_ref[...]
        xi_ref[...] = xi
        xi_s[pl.ds(b * BLK, BLK), :] = xi
        xf_s[pl.ds(b * BLK, BLK), :] = xf
        sv = jnp.dot(xf, wa_ref[...], preferred_element_type=jnp.float32)
        sv = jnp.where(sv > 0, sv, NEG_SLOPE * sv)
        sv_s[pl.ds(b * BLK, BLK), :] = sv
        m = jnp.max(sv)

        @pl.when(b == 0)
        def _():
            gmax_s[0] = m

        @pl.when(b > 0)
        def _():
            gmax_s[0] = jnp.maximum(gmax_s[0], m)

    @pl.when(p == 1)
    def _():
        xi_ref[...] = xi_s[pl.ds(b * BLK, BLK), :]
        esv = jnp.exp(sv_s[pl.ds(b * BLK, BLK), :] - gmax_s[0])
        g_ref[...] = xf_s[pl.ds(b * BLK, BLK), :] * esv
        esv_ref[...] = esv


def _tc1(x_pad, w_x, w_vertex, w_att, bx2, bv2):
    return pl.pallas_call(
        _tc1_body,
        grid=(2, NBN),
        in_specs=[
            pl.BlockSpec((BLK, DIN), lambda p, b: ((1 - p) * b, 0)),
            pl.BlockSpec((DIN, DOUT), lambda p, b: (0, 0)),
            pl.BlockSpec((DIN, DOUT), lambda p, b: (0, 0)),
            pl.BlockSpec((DOUT, 1), lambda p, b: (0, 0)),
            pl.BlockSpec((1, DOUT), lambda p, b: (0, 0)),
            pl.BlockSpec((1, DOUT), lambda p, b: (0, 0)),
        ],
        out_specs=[
            pl.BlockSpec((BLK, DOUT), lambda p, b: (b, 0)),
            pl.BlockSpec((BLK, DOUT), lambda p, b: (b, 0)),
            pl.BlockSpec((BLK, 1), lambda p, b: (b, 0)),
        ],
        out_shape=[
            jax.ShapeDtypeStruct((NP, DOUT), jnp.float32),
            jax.ShapeDtypeStruct((NP, DOUT), jnp.float32),
            jax.ShapeDtypeStruct((NP, 1), jnp.float32),
        ],
        scratch_shapes=[
            pltpu.VMEM((NP, DOUT), jnp.float32),
            pltpu.VMEM((NP, DOUT), jnp.float32),
            pltpu.VMEM((NP, 1), jnp.float32),
            pltpu.SMEM((1,), jnp.float32),
        ],
    )(x_pad, w_x, w_vertex, w_att, bx2, bv2)


def _tc2_body(np_ref, dp_ref, s2_ref, w1_ref, w2_ref, be_ref, y_ref):
    num = np_ref[0] + np_ref[1]
    den = jnp.maximum(dp_ref[0] + dp_ref[1], 1e-12)
    yv = _elu(num / den[:, None])
    y_ref[...] = (
        jnp.dot(yv, w1_ref[...], preferred_element_type=jnp.float32)
        + jnp.dot(s2_ref[...], w2_ref[...], preferred_element_type=jnp.float32)
        + be_ref[...]
    )


def _tc2(num_p, den_p, s2, w1, w2, be2):
    return pl.pallas_call(
        _tc2_body,
        grid=(NBM,),
        in_specs=[
            pl.BlockSpec((NC, BLK, DOUT), lambda b: (0, b, 0)),
            pl.BlockSpec((NC, BLK), lambda b: (0, b)),
            pl.BlockSpec((BLK, DOUT), lambda b: (b, 0)),
            pl.BlockSpec((DOUT, DOUT), lambda b: (0, 0)),
            pl.BlockSpec((DOUT, DOUT), lambda b: (0, 0)),
            pl.BlockSpec((1, DOUT), lambda b: (0, 0)),
        ],
        out_specs=pl.BlockSpec((BLK, DOUT), lambda b: (b, 0)),
        out_shape=jax.ShapeDtypeStruct((MP, DOUT), jnp.float32),
    )(num_p, den_p, s2, w1, w2, be2)


def _tc3_body(xp_ref, cp_ref, xi_ref, out_ref):
    xs = xp_ref[0] + xp_ref[1]
    cnt = jnp.maximum(cp_ref[0] + cp_ref[1], 1.0)
    out_ref[...] = _elu(xs / cnt[:, None]) + xi_ref[...]


def _tc3(xs_p, cnt_p, x_init):
    return pl.pallas_call(
        _tc3_body,
        grid=(NBN,),
        in_specs=[
            pl.BlockSpec((NC, BLK, DOUT), lambda b: (0, b, 0)),
            pl.BlockSpec((NC, BLK), lambda b: (0, b)),
            pl.BlockSpec((BLK, DOUT), lambda b: (b, 0)),
        ],
        out_specs=pl.BlockSpec((BLK, DOUT), lambda b: (b, 0)),
        out_shape=jax.ShapeDtypeStruct((NP, DOUT), jnp.float32),
    )(xs_p, cnt_p, x_init)


_MESH = plsc.VectorSubcoreMesh(core_axis_name="c", subcore_axis_name="s")

_Z16 = functools.partial(jnp.zeros, (LANES,), jnp.float32)


def _zero_1d(ref, n):
    def body(i, _):
        ref[pl.ds(i * LANES, LANES)] = _Z16()
        return 0
    lax.fori_loop(0, n // LANES, body, 0)


def _zero_rows(ref, rows):
    def body(i, _):
        for k in range(DOUT // LANES):
            ref[i, pl.ds(k * LANES, LANES)] = _Z16()
        return 0
    lax.fori_loop(0, rows, body, 0)


@functools.partial(
    pl.kernel,
    out_type=[
        jax.ShapeDtypeStruct((NC, MP, DOUT), jnp.float32),
        jax.ShapeDtypeStruct((NC, MP), jnp.float32),
        jax.ShapeDtypeStruct((NC, NP), jnp.float32),
    ],
    mesh=_MESH,
    scratch_types=[
        pltpu.VMEM((PC, CHUNK), jnp.int32),
        pltpu.VMEM((PC, CHUNK), jnp.int32),
        pltpu.VMEM((NP,), jnp.float32),
        pltpu.VMEM((2, CHUNK, DOUT), jnp.float32),
        pltpu.VMEM((MP,), jnp.float32),
        pltpu.VMEM((NP,), jnp.float32),
        pltpu.VMEM((NS, MPS), jnp.float32),
        pltpu.VMEM((NS, NPS), jnp.float32),
        pltpu.VMEM((MPS,), jnp.float32),
        pltpu.VMEM((NPS,), jnp.float32),
        pltpu.VMEM_SHARED((MP, DOUT), jnp.float32),
        pltpu.VMEM_SHARED((NS, MP), jnp.float32),
        pltpu.VMEM_SHARED((NS, NP), jnp.float32),
        pltpu.SemaphoreType.DMA,
        pltpu.SemaphoreType.DMA,
    ],
    compiler_params=pltpu.CompilerParams(use_tc_tiling_on_sc=False, needs_layout_passes=False),
    name="sc1_v2e",
)
def _sc1(g_hbm, esv_hbm, src_hbm, dst_hbm, num_out, den_out, cnt_out,
         src_v, dst_v, esv_v, rowbuf, den_loc, cnt_loc,
         denred, cntred, denacc, cntacc, num_sh, den_sh, cnt_sh, sem0, sem1):
    cid = lax.axis_index("c")
    sid = lax.axis_index("s")
    wid = cid * NS + sid

    pltpu.sync_copy(src_hbm.at[wid], src_v)
    pltpu.sync_copy(dst_hbm.at[wid], dst_v)
    pltpu.sync_copy(esv_hbm, esv_v)

    _zero_rows(rowbuf.at[0], CHUNK)
    _zero_rows(rowbuf.at[1], CHUNK)
    _zero_1d(den_loc, MP)
    _zero_1d(cnt_loc, NP)
    pltpu.sync_copy(rowbuf.at[0], num_sh.at[pl.ds(sid * MPS, CHUNK)])
    pltpu.sync_copy(rowbuf.at[1], num_sh.at[pl.ds(sid * MPS + CHUNK, CHUNK)])
    pltpu.sync_copy(rowbuf.at[0, pl.ds(0, MPS - 2 * CHUNK)],
                    num_sh.at[pl.ds(sid * MPS + 2 * CHUNK, MPS - 2 * CHUNK)])
    plsc.subcore_barrier()

    pltpu.async_copy(g_hbm.at[src_v.at[0]], rowbuf.at[0], sem0)

    ones16 = jnp.ones((LANES,), jnp.float32)
    sems = (sem0, sem1)

    def chunk_work(jj, b):
        sem = sems[b]
        osem = sems[1 - b]
        pltpu.make_async_copy(g_hbm.at[src_v.at[jj]], rowbuf.at[b], sem).wait()

        @pl.when(jj + 1 < PC)
        def _():
            pltpu.async_copy(g_hbm.at[src_v.at[jj + 1]], rowbuf.at[1 - b], osem)

        desc = pltpu.async_copy(rowbuf.at[b], num_sh.at[dst_v.at[jj]], sem,
                                add=True)
        for k in range(CHUNK // LANES):
            sidx = src_v[jj, pl.ds(k * LANES, LANES)]
            didx = dst_v[jj, pl.ds(k * LANES, LANES)]
            e = plsc.load_gather(esv_v, [sidx])
            plsc.addupdate_scatter(den_loc, [didx], e)
            plsc.addupdate_scatter(cnt_loc, [sidx], ones16)
        desc.wait()

    def body(jh, _):
        chunk_work(2 * jh, 0)
        chunk_work(2 * jh + 1, 1)
        return 0

    lax.fori_loop(0, PC // 2, body, 0)

    plsc.subcore_barrier()
    pltpu.sync_copy(den_loc, den_sh.at[sid])
    pltpu.sync_copy(cnt_loc, cnt_sh.at[sid])
    plsc.subcore_barrier()

    pltpu.sync_copy(num_sh.at[pl.ds(sid * MPS, MPS)],
                    num_out.at[cid, pl.ds(sid * MPS, MPS)])

    for r in range(NS):
        pltpu.sync_copy(den_sh.at[r, pl.ds(sid * MPS, MPS)], denred.at[r])

    def dred(i, _):
        acc = _Z16()
        for r in range(NS):
            acc = acc + denred[r, pl.ds(i * LANES, LANES)]
        denacc[pl.ds(i * LANES, LANES)] = acc
        return 0

    lax.fori_loop(0, MPS // LANES, dred, 0)
    pltpu.sync_copy(denacc, den_out.at[cid, pl.ds(sid * MPS, MPS)])

    for r in range(NS):
        pltpu.sync_copy(cnt_sh.at[r, pl.ds(sid * NPS, NPS)], cntred.at[r])

    def cred(i, _):
        acc = _Z16()
        for r in range(NS):
            acc = acc + cntred[r, pl.ds(i * LANES, LANES)]
        cntacc[pl.ds(i * LANES, LANES)] = acc
        return 0

    lax.fori_loop(0, NPS // LANES, cred, 0)
    pltpu.sync_copy(cntacc, cnt_out.at[cid, pl.ds(sid * NPS, NPS)])


@functools.partial(
    pl.kernel,
    out_type=jax.ShapeDtypeStruct((NC, NP, DOUT), jnp.float32),
    mesh=_MESH,
    scratch_types=[
        pltpu.VMEM((PC, CHUNK), jnp.int32),
        pltpu.VMEM((PC, CHUNK), jnp.int32),
        pltpu.VMEM((2, CHUNK, DOUT), jnp.float32),
        pltpu.VMEM((NPS, DOUT), jnp.float32),
        pltpu.VMEM_SHARED((NP, DOUT), jnp.float32),
        pltpu.SemaphoreType.DMA,
        pltpu.SemaphoreType.DMA,
    ],
    compiler_params=pltpu.CompilerParams(use_tc_tiling_on_sc=False, needs_layout_passes=False),
    name="sc2_e2v",
)
def _sc2(y_hbm, src_hbm, dst_hbm, xs_out,
         src_v, dst_v, rowbuf, zrows, xacc, sem0, sem1):
    cid = lax.axis_index("c")
    sid = lax.axis_index("s")
    wid = cid * NS + sid

    pltpu.sync_copy(src_hbm.at[wid], src_v)
    pltpu.sync_copy(dst_hbm.at[wid], dst_v)

    _zero_rows(zrows, NPS)
    pltpu.sync_copy(zrows, xacc.at[pl.ds(sid * NPS, NPS)])
    plsc.subcore_barrier()

    pltpu.async_copy(y_hbm.at[dst_v.at[0]], rowbuf.at[0], sem0)
    sems = (sem0, sem1)

    def chunk_work(jj, b):
        sem = sems[b]
        osem = sems[1 - b]
        pltpu.make_async_copy(y_hbm.at[dst_v.at[jj]], rowbuf.at[b], sem).wait()

        @pl.when(jj + 1 < PC)
        def _():
            pltpu.async_copy(y_hbm.at[dst_v.at[jj + 1]], rowbuf.at[1 - b], osem)

        pltpu.async_copy(rowbuf.at[b], xacc.at[src_v.at[jj]], sem,
                         add=True).wait()

    def body(jh, _):
        chunk_work(2 * jh, 0)
        chunk_work(2 * jh + 1, 1)
        return 0

    lax.fori_loop(0, PC // 2, body, 0)

    plsc.subcore_barrier()
    pltpu.sync_copy(xacc.at[pl.ds(sid * NPS, NPS)],
                    xs_out.at[cid, pl.ds(sid * NPS, NPS)])


def kernel(X, v2e_src, v2e_dst, S_features, W_x, b_x, W_vertex, b_vertex,
           W_group, b_group, W_att, W_e2v, b_e2v):
    x_pad = jnp.pad(X, ((0, NP - N), (0, 0)))
    npad = NNZP - NNZ
    pad_src = (N + jnp.arange(npad, dtype=jnp.int32) % (NP - N)).astype(jnp.int32)
    pad_dst = (M + jnp.arange(npad, dtype=jnp.int32) % (MP - M)).astype(jnp.int32)
    src_all = jnp.concatenate([v2e_src, pad_src])
    dst_all = jnp.concatenate([v2e_dst, pad_dst])
    src_t = src_all.reshape(NW, PC, CHUNK)
    dst_t = dst_all.reshape(NW, PC, CHUNK)
    src_s = jnp.swapaxes(src_all.reshape(NW, CHUNK, PC), 1, 2)
    dst_s = jnp.swapaxes(dst_all.reshape(NW, CHUNK, PC), 1, 2)

    s2 = jnp.pad(S_features, ((0, MP - M), (0, DOUT - DS)))
    w1 = W_e2v[:DOUT]
    w2 = jnp.pad(W_e2v[DOUT:], ((0, DOUT - DS), (0, 0)))
    bx2 = b_x[None, :]
    bv2 = b_vertex[None, :]
    be2 = b_e2v[None, :]

    x_init, g, esv2 = _tc1(x_pad, W_x, W_vertex, W_att, bx2, bv2)
    esv = esv2.reshape(NP)

    num_p, den_p, cnt_p = _sc1(g, esv, src_t, dst_t)
    y = _tc2(num_p, den_p, s2, w1, w2, be2)
    xs_p = _sc2(y, src_s, dst_s)
    out = _tc3(xs_p, cnt_p, x_init)
    return out[:N]


# speedup vs baseline: 1.5213x; 1.3314x over previous
import functools

import jax
import jax.numpy as jnp
from jax import lax
from jax.experimental import pallas as pl
from jax.experimental.pallas import tpu as pltpu
from jax.experimental.pallas import tpu_sc as plsc

N = 10000
M = 5000
NNZ = 320000
DIN = 128
DOUT = 64
DS = 10
NEG_SLOPE = 0.2

NC = 2
NS = 16
NW = NC * NS
LANES = 16

NP = 10240
MP = 5120
NPS = NP // NS
MPS = MP // NS

CHUNK = 128
PC = 80
PT = PC * CHUNK
NNZP = NW * PT

BLK = 256
NBN = NP // BLK
NBM = MP // BLK


def _elu(x):
    return jnp.where(x > 0, x, jnp.exp(jnp.minimum(x, 0.0)) - 1.0)


def _tc1_body(x_ref, wx_ref, wv_ref, wa_ref, bx_ref, bv_ref,
              xi_ref, g_ref, esv_ref, xi_s, xf_s, sv_s, gmax_s):
    p = pl.program_id(0)
    b = pl.program_id(1)

    @pl.when(p == 0)
    def _():
        x = x_ref[...]
        xf = jnp.dot(x, wv_ref[...], preferred_element_type=jnp.float32) + bv_ref[...]
        xi = jnp.dot(x, wx_ref[...], preferred_element_type=jnp.float32) + bx_ref[...]
        xi_ref[...] = xi
        xi_s[pl.ds(b * BLK, BLK), :] = xi
        xf_s[pl.ds(b * BLK, BLK), :] = xf
        sv = jnp.dot(xf, wa_ref[...], preferred_element_type=jnp.float32)
        sv = jnp.where(sv > 0, sv, NEG_SLOPE * sv)
        sv_s[pl.ds(b * BLK, BLK), :] = sv
        m = jnp.max(sv)

        @pl.when(b == 0)
        def _():
            gmax_s[0] = m

        @pl.when(b > 0)
        def _():
            gmax_s[0] = jnp.maximum(gmax_s[0], m)

    @pl.when(p == 1)
    def _():
        xi_ref[...] = xi_s[pl.ds(b * BLK, BLK), :]
        esv = jnp.exp(sv_s[pl.ds(b * BLK, BLK), :] - gmax_s[0])
        g_ref[...] = xf_s[pl.ds(b * BLK, BLK), :] * esv
        esv_ref[...] = esv


def _tc1(x_pad, w_x, w_vertex, w_att, bx2, bv2):
    return pl.pallas_call(
        _tc1_body,
        grid=(2, NBN),
        in_specs=[
            pl.BlockSpec((BLK, DIN), lambda p, b: ((1 - p) * b, 0)),
            pl.BlockSpec((DIN, DOUT), lambda p, b: (0, 0)),
            pl.BlockSpec((DIN, DOUT), lambda p, b: (0, 0)),
            pl.BlockSpec((DOUT, 1), lambda p, b: (0, 0)),
            pl.BlockSpec((1, DOUT), lambda p, b: (0, 0)),
            pl.BlockSpec((1, DOUT), lambda p, b: (0, 0)),
        ],
        out_specs=[
            pl.BlockSpec((BLK, DOUT), lambda p, b: (b, 0)),
            pl.BlockSpec((BLK, DOUT), lambda p, b: (b, 0)),
            pl.BlockSpec((BLK, 1), lambda p, b: (b, 0)),
        ],
        out_shape=[
            jax.ShapeDtypeStruct((NP, DOUT), jnp.float32),
            jax.ShapeDtypeStruct((NP, DOUT), jnp.float32),
            jax.ShapeDtypeStruct((NP, 1), jnp.float32),
        ],
        scratch_shapes=[
            pltpu.VMEM((NP, DOUT), jnp.float32),
            pltpu.VMEM((NP, DOUT), jnp.float32),
            pltpu.VMEM((NP, 1), jnp.float32),
            pltpu.SMEM((1,), jnp.float32),
        ],
    )(x_pad, w_x, w_vertex, w_att, bx2, bv2)


def _tc2_body(np_ref, dp_ref, s2_ref, w1_ref, w2_ref, be_ref, y_ref):
    num = np_ref[0] + np_ref[1]
    den = jnp.maximum(dp_ref[0] + dp_ref[1], 1e-12)
    yv = _elu(num / den[:, None])
    y_ref[...] = (
        jnp.dot(yv, w1_ref[...], preferred_element_type=jnp.float32)
        + jnp.dot(s2_ref[...], w2_ref[...], preferred_element_type=jnp.float32)
        + be_ref[...]
    )


def _tc2(num_p, den_p, s2, w1, w2, be2):
    return pl.pallas_call(
        _tc2_body,
        grid=(NBM,),
        in_specs=[
            pl.BlockSpec((NC, BLK, DOUT), lambda b: (0, b, 0)),
            pl.BlockSpec((NC, BLK), lambda b: (0, b)),
            pl.BlockSpec((BLK, DOUT), lambda b: (b, 0)),
            pl.BlockSpec((DOUT, DOUT), lambda b: (0, 0)),
            pl.BlockSpec((DOUT, DOUT), lambda b: (0, 0)),
            pl.BlockSpec((1, DOUT), lambda b: (0, 0)),
        ],
        out_specs=pl.BlockSpec((BLK, DOUT), lambda b: (b, 0)),
        out_shape=jax.ShapeDtypeStruct((MP, DOUT), jnp.float32),
    )(num_p, den_p, s2, w1, w2, be2)


def _tc3_body(xp_ref, cp_ref, xi_ref, out_ref):
    xs = xp_ref[0] + xp_ref[1]
    cnt = jnp.maximum(cp_ref[0] + cp_ref[1], 1.0)
    out_ref[...] = _elu(xs / cnt[:, None]) + xi_ref[...]


def _tc3(xs_p, cnt_p, x_init):
    return pl.pallas_call(
        _tc3_body,
        grid=(NBN,),
        in_specs=[
            pl.BlockSpec((NC, BLK, DOUT), lambda b: (0, b, 0)),
            pl.BlockSpec((NC, BLK), lambda b: (0, b)),
            pl.BlockSpec((BLK, DOUT), lambda b: (b, 0)),
        ],
        out_specs=pl.BlockSpec((BLK, DOUT), lambda b: (b, 0)),
        out_shape=jax.ShapeDtypeStruct((NP, DOUT), jnp.float32),
    )(xs_p, cnt_p, x_init)


_MESH = plsc.VectorSubcoreMesh(core_axis_name="c", subcore_axis_name="s")

_Z16 = functools.partial(jnp.zeros, (LANES,), jnp.float32)


def _zero_1d(ref, n):
    def body(i, _):
        ref[pl.ds(i * LANES, LANES)] = _Z16()
        return 0
    lax.fori_loop(0, n // LANES, body, 0)


def _zero_rows(ref, rows):
    def body(i, _):
        for k in range(DOUT // LANES):
            ref[i, pl.ds(k * LANES, LANES)] = _Z16()
        return 0
    lax.fori_loop(0, rows, body, 0)


@functools.partial(
    pl.kernel,
    out_type=[
        jax.ShapeDtypeStruct((NC, MP, DOUT), jnp.float32),
        jax.ShapeDtypeStruct((NC, MP), jnp.float32),
        jax.ShapeDtypeStruct((NC, NP), jnp.float32),
    ],
    mesh=_MESH,
    scratch_types=[
        pltpu.VMEM((PC, CHUNK), jnp.int32),
        pltpu.VMEM((PC, CHUNK), jnp.int32),
        pltpu.VMEM((NP,), jnp.float32),
        pltpu.VMEM((4, CHUNK, DOUT), jnp.float32),
        pltpu.VMEM((MP,), jnp.float32),
        pltpu.VMEM((NP,), jnp.float32),
        pltpu.VMEM((NS, MPS), jnp.float32),
        pltpu.VMEM((NS, NPS), jnp.float32),
        pltpu.VMEM((MPS,), jnp.float32),
        pltpu.VMEM((NPS,), jnp.float32),
        pltpu.VMEM_SHARED((MP, DOUT), jnp.float32),
        pltpu.VMEM_SHARED((NS, MP), jnp.float32),
        pltpu.VMEM_SHARED((NS, NP), jnp.float32),
        pltpu.SemaphoreType.DMA,
        pltpu.SemaphoreType.DMA,
        pltpu.SemaphoreType.DMA,
        pltpu.SemaphoreType.DMA,
        pltpu.SemaphoreType.DMA,
        pltpu.SemaphoreType.DMA,
        pltpu.SemaphoreType.DMA,
        pltpu.SemaphoreType.DMA,
    ],
    compiler_params=pltpu.CompilerParams(use_tc_tiling_on_sc=False, needs_layout_passes=False),
    name="sc1_v2e",
)
def _sc1(g_hbm, esv_hbm, src_hbm, dst_hbm, num_out, den_out, cnt_out,
         src_v, dst_v, esv_v, rowbuf, den_loc, cnt_loc,
         denred, cntred, denacc, cntacc, num_sh, den_sh, cnt_sh,
         gs0, gs1, gs2, gs3, ss0, ss1, ss2, ss3):
    cid = lax.axis_index("c")
    sid = lax.axis_index("s")
    wid = cid * NS + sid

    pltpu.sync_copy(src_hbm.at[wid], src_v)
    pltpu.sync_copy(dst_hbm.at[wid], dst_v)
    pltpu.sync_copy(esv_hbm, esv_v)

    _zero_rows(rowbuf.at[0], CHUNK)
    _zero_rows(rowbuf.at[1], CHUNK)
    _zero_1d(den_loc, MP)
    _zero_1d(cnt_loc, NP)
    pltpu.sync_copy(rowbuf.at[0], num_sh.at[pl.ds(sid * MPS, CHUNK)])
    pltpu.sync_copy(rowbuf.at[1], num_sh.at[pl.ds(sid * MPS + CHUNK, CHUNK)])
    pltpu.sync_copy(rowbuf.at[0, pl.ds(0, MPS - 2 * CHUNK)],
                    num_sh.at[pl.ds(sid * MPS + 2 * CHUNK, MPS - 2 * CHUNK)])
    plsc.subcore_barrier()

    ones16 = jnp.ones((LANES,), jnp.float32)
    gsems = (gs0, gs1, gs2, gs3)
    ssems = (ss0, ss1, ss2, ss3)
    KR = 4

    for b in range(KR - 1):
        pltpu.async_copy(g_hbm.at[src_v.at[b]], rowbuf.at[b], gsems[b])

    def chunk_work(jh, b):
        jj = jh * KR + b
        pb = (b - 1) % KR

        def wait_prev_scatter():
            pltpu.make_async_copy(rowbuf.at[pb],
                                  num_sh.at[dst_v.at[0]], ssems[pb]).wait()
        if b == 0:
            @pl.when(jh > 0)
            def _():
                wait_prev_scatter()
        else:
            wait_prev_scatter()

        @pl.when(jj + (KR - 1) < PC)
        def _():
            pltpu.async_copy(g_hbm.at[src_v.at[jj + KR - 1]],
                             rowbuf.at[pb], gsems[pb])

        pltpu.make_async_copy(g_hbm.at[src_v.at[jj]], rowbuf.at[b],
                              gsems[b]).wait()
        pltpu.async_copy(rowbuf.at[b], num_sh.at[dst_v.at[jj]], ssems[b],
                         add=True)
        for k in range(CHUNK // LANES):
            sidx = src_v[jj, pl.ds(k * LANES, LANES)]
            didx = dst_v[jj, pl.ds(k * LANES, LANES)]
            e = plsc.load_gather(esv_v, [sidx])
            plsc.addupdate_scatter(den_loc, [didx], e)
            plsc.addupdate_scatter(cnt_loc, [sidx], ones16)

    def body(jh, _):
        for b in range(KR):
            chunk_work(jh, b)
        return 0

    lax.fori_loop(0, PC // KR, body, 0)
    pltpu.make_async_copy(rowbuf.at[KR - 1], num_sh.at[dst_v.at[0]],
                          ssems[KR - 1]).wait()

    plsc.subcore_barrier()
    pltpu.sync_copy(den_loc, den_sh.at[sid])
    pltpu.sync_copy(cnt_loc, cnt_sh.at[sid])
    plsc.subcore_barrier()

    pltpu.sync_copy(num_sh.at[pl.ds(sid * MPS, MPS)],
                    num_out.at[cid, pl.ds(sid * MPS, MPS)])

    for r in range(NS):
        pltpu.sync_copy(den_sh.at[r, pl.ds(sid * MPS, MPS)], denred.at[r])

    def dred(i, _):
        acc = _Z16()
        for r in range(NS):
            acc = acc + denred[r, pl.ds(i * LANES, LANES)]
        denacc[pl.ds(i * LANES, LANES)] = acc
        return 0

    lax.fori_loop(0, MPS // LANES, dred, 0)
    pltpu.sync_copy(denacc, den_out.at[cid, pl.ds(sid * MPS, MPS)])

    for r in range(NS):
        pltpu.sync_copy(cnt_sh.at[r, pl.ds(sid * NPS, NPS)], cntred.at[r])

    def cred(i, _):
        acc = _Z16()
        for r in range(NS):
            acc = acc + cntred[r, pl.ds(i * LANES, LANES)]
        cntacc[pl.ds(i * LANES, LANES)] = acc
        return 0

    lax.fori_loop(0, NPS // LANES, cred, 0)
    pltpu.sync_copy(cntacc, cnt_out.at[cid, pl.ds(sid * NPS, NPS)])


@functools.partial(
    pl.kernel,
    out_type=jax.ShapeDtypeStruct((NC, NP, DOUT), jnp.float32),
    mesh=_MESH,
    scratch_types=[
        pltpu.VMEM((PC, CHUNK), jnp.int32),
        pltpu.VMEM((PC, CHUNK), jnp.int32),
        pltpu.VMEM((4, CHUNK, DOUT), jnp.float32),
        pltpu.VMEM_SHARED((NP, DOUT), jnp.float32),
        pltpu.SemaphoreType.DMA,
        pltpu.SemaphoreType.DMA,
        pltpu.SemaphoreType.DMA,
        pltpu.SemaphoreType.DMA,
        pltpu.SemaphoreType.DMA,
        pltpu.SemaphoreType.DMA,
        pltpu.SemaphoreType.DMA,
        pltpu.SemaphoreType.DMA,
    ],
    compiler_params=pltpu.CompilerParams(use_tc_tiling_on_sc=False, needs_layout_passes=False),
    name="sc2_e2v",
)
def _sc2(y_hbm, src_hbm, dst_hbm, xs_out,
         src_v, dst_v, rowbuf, xacc,
         gs0, gs1, gs2, gs3, ss0, ss1, ss2, ss3):
    cid = lax.axis_index("c")
    sid = lax.axis_index("s")
    wid = cid * NS + sid

    pltpu.sync_copy(src_hbm.at[wid], src_v)
    pltpu.sync_copy(dst_hbm.at[wid], dst_v)

    for b in range(4):
        _zero_rows(rowbuf.at[b], CHUNK)
    for j in range(NPS // CHUNK):
        pltpu.sync_copy(rowbuf.at[j % 4],
                        xacc.at[pl.ds(sid * NPS + j * CHUNK, CHUNK)])
    plsc.subcore_barrier()

    gsems = (gs0, gs1, gs2, gs3)
    ssems = (ss0, ss1, ss2, ss3)
    KR = 4

    for b in range(KR - 1):
        pltpu.async_copy(y_hbm.at[dst_v.at[b]], rowbuf.at[b], gsems[b])

    def chunk_work(jh, b):
        jj = jh * KR + b
        pb = (b - 1) % KR

        def wait_prev_scatter():
            pltpu.make_async_copy(rowbuf.at[pb],
                                  xacc.at[src_v.at[0]], ssems[pb]).wait()
        if b == 0:
            @pl.when(jh > 0)
            def _():
                wait_prev_scatter()
        else:
            wait_prev_scatter()

        @pl.when(jj + (KR - 1) < PC)
        def _():
            pltpu.async_copy(y_hbm.at[dst_v.at[jj + KR - 1]],
                             rowbuf.at[pb], gsems[pb])

        pltpu.make_async_copy(y_hbm.at[dst_v.at[jj]], rowbuf.at[b],
                              gsems[b]).wait()
        pltpu.async_copy(rowbuf.at[b], xacc.at[src_v.at[jj]], ssems[b],
                         add=True)

    def body(jh, _):
        for b in range(KR):
            chunk_work(jh, b)
        return 0

    lax.fori_loop(0, PC // KR, body, 0)
    pltpu.make_async_copy(rowbuf.at[KR - 1], xacc.at[src_v.at[0]],
                          ssems[KR - 1]).wait()

    plsc.subcore_barrier()
    pltpu.sync_copy(xacc.at[pl.ds(sid * NPS, NPS)],
                    xs_out.at[cid, pl.ds(sid * NPS, NPS)])


def kernel(X, v2e_src, v2e_dst, S_features, W_x, b_x, W_vertex, b_vertex,
           W_group, b_group, W_att, W_e2v, b_e2v):
    x_pad = jnp.pad(X, ((0, NP - N), (0, 0)))
    npad = NNZP - NNZ
    pad_src = (N + jnp.arange(npad, dtype=jnp.int32) % (NP - N)).astype(jnp.int32)
    pad_dst = (M + jnp.arange(npad, dtype=jnp.int32) % (MP - M)).astype(jnp.int32)
    src_all = jnp.concatenate([v2e_src, pad_src])
    dst_all = jnp.concatenate([v2e_dst, pad_dst])
    src_t = src_all.reshape(NW, PC, CHUNK)
    dst_t = dst_all.reshape(NW, PC, CHUNK)
    src_s = jnp.swapaxes(src_all.reshape(NW, CHUNK, PC), 1, 2)
    dst_s = jnp.swapaxes(dst_all.reshape(NW, CHUNK, PC), 1, 2)

    s2 = jnp.pad(S_features, ((0, MP - M), (0, DOUT - DS)))
    w1 = W_e2v[:DOUT]
    w2 = jnp.pad(W_e2v[DOUT:], ((0, DOUT - DS), (0, 0)))
    bx2 = b_x[None, :]
    bv2 = b_vertex[None, :]
    be2 = b_e2v[None, :]

    x_init, g, esv2 = _tc1(x_pad, W_x, W_vertex, W_att, bx2, bv2)
    esv = esv2.reshape(NP)

    num_p, den_p, cnt_p = _sc1(g, esv, src_t, dst_t)
    y = _tc2(num_p, den_p, s2, w1, w2, be2)
    xs_p = _sc2(y, src_s, dst_s)
    out = _tc3(xs_p, cnt_p, x_init)
    return out[:N]


# speedup vs baseline: 1.5482x; 1.0177x over previous
import functools

import jax
import jax.numpy as jnp
from jax import lax
from jax.experimental import pallas as pl
from jax.experimental.pallas import tpu as pltpu
from jax.experimental.pallas import tpu_sc as plsc

N = 10000
M = 5000
NNZ = 320000
DIN = 128
DOUT = 64
DS = 10
NEG_SLOPE = 0.2

NC = 2
NS = 16
NW = NC * NS
LANES = 16

NP = 10240
MP = 5120
NPS = NP // NS
MPS = MP // NS

CHUNK = 128
PC = 80
PT = PC * CHUNK
NNZP = NW * PT

CH2 = 256
PC2 = PT // CH2

BLK = 256
NBN = NP // BLK
NBM = MP // BLK


def _elu(x):
    return jnp.where(x > 0, x, jnp.exp(jnp.minimum(x, 0.0)) - 1.0)


def _tc1_body(x_ref, wx_ref, wv_ref, wa_ref, bx_ref, bv_ref,
              xi_ref, g_ref, esv_ref, xi_s, xf_s, sv_s, gmax_s):
    p = pl.program_id(0)
    b = pl.program_id(1)

    @pl.when(p == 0)
    def _():
        x = x_ref[...]
        xf = jnp.dot(x, wv_ref[...], preferred_element_type=jnp.float32) + bv_ref[...]
        xi = jnp.dot(x, wx_ref[...], preferred_element_type=jnp.float32) + bx_ref[...]
        xi_ref[...] = xi
        xi_s[pl.ds(b * BLK, BLK), :] = xi
        xf_s[pl.ds(b * BLK, BLK), :] = xf
        sv = jnp.dot(xf, wa_ref[...], preferred_element_type=jnp.float32)
        sv = jnp.where(sv > 0, sv, NEG_SLOPE * sv)
        sv_s[pl.ds(b * BLK, BLK), :] = sv
        m = jnp.max(sv)

        @pl.when(b == 0)
        def _():
            gmax_s[0] = m

        @pl.when(b > 0)
        def _():
            gmax_s[0] = jnp.maximum(gmax_s[0], m)

    @pl.when(p == 1)
    def _():
        xi_ref[...] = xi_s[pl.ds(b * BLK, BLK), :]
        esv = jnp.exp(sv_s[pl.ds(b * BLK, BLK), :] - gmax_s[0])
        g_ref[...] = xf_s[pl.ds(b * BLK, BLK), :] * esv
        esv_ref[...] = esv


def _tc1(x_pad, w_x, w_vertex, w_att, bx2, bv2):
    return pl.pallas_call(
        _tc1_body,
        grid=(2, NBN),
        in_specs=[
            pl.BlockSpec((BLK, DIN), lambda p, b: ((1 - p) * b, 0)),
            pl.BlockSpec((DIN, DOUT), lambda p, b: (0, 0)),
            pl.BlockSpec((DIN, DOUT), lambda p, b: (0, 0)),
            pl.BlockSpec((DOUT, 1), lambda p, b: (0, 0)),
            pl.BlockSpec((1, DOUT), lambda p, b: (0, 0)),
            pl.BlockSpec((1, DOUT), lambda p, b: (0, 0)),
        ],
        out_specs=[
            pl.BlockSpec((BLK, DOUT), lambda p, b: (b, 0)),
            pl.BlockSpec((BLK, DOUT), lambda p, b: (b, 0)),
            pl.BlockSpec((BLK, 1), lambda p, b: (b, 0)),
        ],
        out_shape=[
            jax.ShapeDtypeStruct((NP, DOUT), jnp.float32),
            jax.ShapeDtypeStruct((NP, DOUT), jnp.float32),
            jax.ShapeDtypeStruct((NP, 1), jnp.float32),
        ],
        scratch_shapes=[
            pltpu.VMEM((NP, DOUT), jnp.float32),
            pltpu.VMEM((NP, DOUT), jnp.float32),
            pltpu.VMEM((NP, 1), jnp.float32),
            pltpu.SMEM((1,), jnp.float32),
        ],
    )(x_pad, w_x, w_vertex, w_att, bx2, bv2)


def _tc2_body(np_ref, dp_ref, s2_ref, w1_ref, w2_ref, be_ref, y_ref):
    num = np_ref[0] + np_ref[1]
    den = jnp.maximum(dp_ref[0] + dp_ref[1], 1e-12)
    yv = _elu(num / den[:, None])
    y_ref[...] = (
        jnp.dot(yv, w1_ref[...], preferred_element_type=jnp.float32)
        + jnp.dot(s2_ref[...], w2_ref[...], preferred_element_type=jnp.float32)
        + be_ref[...]
    )


def _tc2(num_p, den_p, s2, w1, w2, be2):
    return pl.pallas_call(
        _tc2_body,
        grid=(NBM,),
        in_specs=[
            pl.BlockSpec((NC, BLK, DOUT), lambda b: (0, b, 0)),
            pl.BlockSpec((NC, BLK), lambda b: (0, b)),
            pl.BlockSpec((BLK, DOUT), lambda b: (b, 0)),
            pl.BlockSpec((DOUT, DOUT), lambda b: (0, 0)),
            pl.BlockSpec((DOUT, DOUT), lambda b: (0, 0)),
            pl.BlockSpec((1, DOUT), lambda b: (0, 0)),
        ],
        out_specs=pl.BlockSpec((BLK, DOUT), lambda b: (b, 0)),
        out_shape=jax.ShapeDtypeStruct((MP, DOUT), jnp.float32),
    )(num_p, den_p, s2, w1, w2, be2)


def _tc3_body(xp_ref, cp_ref, xi_ref, out_ref):
    xs = xp_ref[0] + xp_ref[1]
    cnt = jnp.maximum(cp_ref[0] + cp_ref[1], 1.0)
    out_ref[...] = _elu(xs / cnt[:, None]) + xi_ref[...]


def _tc3(xs_p, cnt_p, x_init):
    return pl.pallas_call(
        _tc3_body,
        grid=(NBN,),
        in_specs=[
            pl.BlockSpec((NC, BLK, DOUT), lambda b: (0, b, 0)),
            pl.BlockSpec((NC, BLK), lambda b: (0, b)),
            pl.BlockSpec((BLK, DOUT), lambda b: (b, 0)),
        ],
        out_specs=pl.BlockSpec((BLK, DOUT), lambda b: (b, 0)),
        out_shape=jax.ShapeDtypeStruct((NP, DOUT), jnp.float32),
    )(xs_p, cnt_p, x_init)


_MESH = plsc.VectorSubcoreMesh(core_axis_name="c", subcore_axis_name="s")

_Z16 = functools.partial(jnp.zeros, (LANES,), jnp.float32)


def _zero_1d(ref, n):
    def body(i, _):
        ref[pl.ds(i * LANES, LANES)] = _Z16()
        return 0
    lax.fori_loop(0, n // LANES, body, 0)


def _zero_rows(ref, rows):
    def body(i, _):
        for k in range(DOUT // LANES):
            ref[i, pl.ds(k * LANES, LANES)] = _Z16()
        return 0
    lax.fori_loop(0, rows, body, 0)


@functools.partial(
    pl.kernel,
    out_type=[
        jax.ShapeDtypeStruct((NC, MP, DOUT), jnp.float32),
        jax.ShapeDtypeStruct((NC, MP), jnp.float32),
        jax.ShapeDtypeStruct((NC, NP), jnp.float32),
    ],
    mesh=_MESH,
    scratch_types=[
        pltpu.VMEM((PC, CHUNK), jnp.int32),
        pltpu.VMEM((PC, CHUNK), jnp.int32),
        pltpu.VMEM((NP,), jnp.float32),
        pltpu.VMEM((4, CHUNK, DOUT), jnp.float32),
        pltpu.VMEM((MP,), jnp.float32),
        pltpu.VMEM((NP,), jnp.float32),
        pltpu.VMEM((NS, MPS), jnp.float32),
        pltpu.VMEM((NS, NPS), jnp.float32),
        pltpu.VMEM((MPS,), jnp.float32),
        pltpu.VMEM((NPS,), jnp.float32),
        pltpu.VMEM_SHARED((MP, DOUT), jnp.float32),
        pltpu.VMEM_SHARED((NS, MP), jnp.float32),
        pltpu.VMEM_SHARED((NS, NP), jnp.float32),
        pltpu.SemaphoreType.DMA,
        pltpu.SemaphoreType.DMA,
        pltpu.SemaphoreType.DMA,
        pltpu.SemaphoreType.DMA,
        pltpu.SemaphoreType.DMA,
        pltpu.SemaphoreType.DMA,
        pltpu.SemaphoreType.DMA,
        pltpu.SemaphoreType.DMA,
    ],
    compiler_params=pltpu.CompilerParams(use_tc_tiling_on_sc=False, needs_layout_passes=False),
    name="sc1_v2e",
)
def _sc1(g_hbm, esv_hbm, src_hbm, dst_hbm, num_out, den_out, cnt_out,
         src_v, dst_v, esv_v, rowbuf, den_loc, cnt_loc,
         denred, cntred, denacc, cntacc, num_sh, den_sh, cnt_sh,
         gs0, gs1, gs2, gs3, ss0, ss1, ss2, ss3):
    cid = lax.axis_index("c")
    sid = lax.axis_index("s")
    wid = cid * NS + sid

    pltpu.sync_copy(src_hbm.at[wid], src_v)
    pltpu.sync_copy(dst_hbm.at[wid], dst_v)
    pltpu.sync_copy(esv_hbm, esv_v)

    _zero_rows(rowbuf.at[0], CHUNK)
    _zero_rows(rowbuf.at[1], CHUNK)
    _zero_1d(den_loc, MP)
    _zero_1d(cnt_loc, NP)
    pltpu.sync_copy(rowbuf.at[0], num_sh.at[pl.ds(sid * MPS, CHUNK)])
    pltpu.sync_copy(rowbuf.at[1], num_sh.at[pl.ds(sid * MPS + CHUNK, CHUNK)])
    pltpu.sync_copy(rowbuf.at[0, pl.ds(0, MPS - 2 * CHUNK)],
                    num_sh.at[pl.ds(sid * MPS + 2 * CHUNK, MPS - 2 * CHUNK)])
    plsc.subcore_barrier()

    ones16 = jnp.ones((LANES,), jnp.float32)
    gsems = (gs0, gs1, gs2, gs3)
    ssems = (ss0, ss1, ss2, ss3)
    KR = 4

    for b in range(KR - 1):
        pltpu.async_copy(g_hbm.at[src_v.at[b]], rowbuf.at[b], gsems[b])

    def chunk_work(jh, b):
        jj = jh * KR + b
        pb = (b - 1) % KR

        def wait_prev_scatter():
            pltpu.make_async_copy(rowbuf.at[pb],
                                  num_sh.at[dst_v.at[0]], ssems[pb]).wait()
        if b == 0:
            @pl.when(jh > 0)
            def _():
                wait_prev_scatter()
        else:
            wait_prev_scatter()

        @pl.when(jj + (KR - 1) < PC)
        def _():
            pltpu.async_copy(g_hbm.at[src_v.at[jj + KR - 1]],
                             rowbuf.at[pb], gsems[pb])

        pltpu.make_async_copy(g_hbm.at[src_v.at[jj]], rowbuf.at[b],
                              gsems[b]).wait()
        pltpu.async_copy(rowbuf.at[b], num_sh.at[dst_v.at[jj]], ssems[b],
                         add=True)
        for k in range(CHUNK // LANES):
            sidx = src_v[jj, pl.ds(k * LANES, LANES)]
            didx = dst_v[jj, pl.ds(k * LANES, LANES)]
            e = plsc.load_gather(esv_v, [sidx])
            plsc.addupdate_scatter(den_loc, [didx], e)
            plsc.addupdate_scatter(cnt_loc, [sidx], ones16)

    def body(jh, _):
        for b in range(KR):
            chunk_work(jh, b)
        return 0

    lax.fori_loop(0, PC // KR, body, 0)
    pltpu.make_async_copy(rowbuf.at[KR - 1], num_sh.at[dst_v.at[0]],
                          ssems[KR - 1]).wait()

    plsc.subcore_barrier()
    pltpu.sync_copy(den_loc, den_sh.at[sid])
    pltpu.sync_copy(cnt_loc, cnt_sh.at[sid])
    plsc.subcore_barrier()

    pltpu.sync_copy(num_sh.at[pl.ds(sid * MPS, MPS)],
                    num_out.at[cid, pl.ds(sid * MPS, MPS)])

    for r in range(NS):
        pltpu.sync_copy(den_sh.at[r, pl.ds(sid * MPS, MPS)], denred.at[r])

    def dred(i, _):
        acc = _Z16()
        for r in range(NS):
            acc = acc + denred[r, pl.ds(i * LANES, LANES)]
        denacc[pl.ds(i * LANES, LANES)] = acc
        return 0

    lax.fori_loop(0, MPS // LANES, dred, 0)
    pltpu.sync_copy(denacc, den_out.at[cid, pl.ds(sid * MPS, MPS)])

    for r in range(NS):
        pltpu.sync_copy(cnt_sh.at[r, pl.ds(sid * NPS, NPS)], cntred.at[r])

    def cred(i, _):
        acc = _Z16()
        for r in range(NS):
            acc = acc + cntred[r, pl.ds(i * LANES, LANES)]
        cntacc[pl.ds(i * LANES, LANES)] = acc
        return 0

    lax.fori_loop(0, NPS // LANES, cred, 0)
    pltpu.sync_copy(cntacc, cnt_out.at[cid, pl.ds(sid * NPS, NPS)])


@functools.partial(
    pl.kernel,
    out_type=jax.ShapeDtypeStruct((NC, NP, DOUT), jnp.float32),
    mesh=_MESH,
    scratch_types=[
        pltpu.VMEM((PC2, CH2), jnp.int32),
        pltpu.VMEM((PC2, CH2), jnp.int32),
        pltpu.VMEM((4, CH2, DOUT), jnp.float32),
        pltpu.VMEM_SHARED((NP, DOUT), jnp.float32),
        pltpu.SemaphoreType.DMA,
        pltpu.SemaphoreType.DMA,
        pltpu.SemaphoreType.DMA,
        pltpu.SemaphoreType.DMA,
        pltpu.SemaphoreType.DMA,
        pltpu.SemaphoreType.DMA,
        pltpu.SemaphoreType.DMA,
        pltpu.SemaphoreType.DMA,
    ],
    compiler_params=pltpu.CompilerParams(use_tc_tiling_on_sc=False, needs_layout_passes=False),
    name="sc2_e2v",
)
def _sc2(y_hbm, src_hbm, dst_hbm, xs_out,
         src_v, dst_v, rowbuf, xacc,
         gs0, gs1, gs2, gs3, ss0, ss1, ss2, ss3):
    cid = lax.axis_index("c")
    sid = lax.axis_index("s")
    wid = cid * NS + sid

    pltpu.sync_copy(src_hbm.at[wid], src_v)
    pltpu.sync_copy(dst_hbm.at[wid], dst_v)

    for b in range(3):
        _zero_rows(rowbuf.at[b], CH2)
    pltpu.sync_copy(rowbuf.at[0], xacc.at[pl.ds(sid * NPS, CH2)])
    pltpu.sync_copy(rowbuf.at[1], xacc.at[pl.ds(sid * NPS + CH2, CH2)])
    pltpu.sync_copy(rowbuf.at[2, pl.ds(0, NPS - 2 * CH2)],
                    xacc.at[pl.ds(sid * NPS + 2 * CH2, NPS - 2 * CH2)])
    plsc.subcore_barrier()

    gsems = (gs0, gs1, gs2, gs3)
    ssems = (ss0, ss1, ss2, ss3)
    KR = 4

    for b in range(KR - 1):
        pltpu.async_copy(y_hbm.at[dst_v.at[b]], rowbuf.at[b], gsems[b])

    def chunk_work(jh, b):
        jj = jh * KR + b
        pb = (b - 1) % KR

        def wait_prev_scatter():
            pltpu.make_async_copy(rowbuf.at[pb],
                                  xacc.at[src_v.at[0]], ssems[pb]).wait()
        if b == 0:
            @pl.when(jh > 0)
            def _():
                wait_prev_scatter()
        else:
            wait_prev_scatter()

        @pl.when(jj + (KR - 1) < PC2)
        def _():
            pltpu.async_copy(y_hbm.at[dst_v.at[jj + KR - 1]],
                             rowbuf.at[pb], gsems[pb])

        pltpu.make_async_copy(y_hbm.at[dst_v.at[jj]], rowbuf.at[b],
                              gsems[b]).wait()
        pltpu.async_copy(rowbuf.at[b], xacc.at[src_v.at[jj]], ssems[b],
                         add=True)

    def body(jh, _):
        for b in range(KR):
            chunk_work(jh, b)
        return 0

    lax.fori_loop(0, PC2 // KR, body, 0)
    pltpu.make_async_copy(rowbuf.at[KR - 1], xacc.at[src_v.at[0]],
                          ssems[KR - 1]).wait()

    plsc.subcore_barrier()
    pltpu.sync_copy(xacc.at[pl.ds(sid * NPS, NPS)],
                    xs_out.at[cid, pl.ds(sid * NPS, NPS)])


def kernel(X, v2e_src, v2e_dst, S_features, W_x, b_x, W_vertex, b_vertex,
           W_group, b_group, W_att, W_e2v, b_e2v):
    x_pad = jnp.pad(X, ((0, NP - N), (0, 0)))
    npad = NNZP - NNZ
    pad_src = (N + jnp.arange(npad, dtype=jnp.int32) % (NP - N)).astype(jnp.int32)
    pad_dst = (M + jnp.arange(npad, dtype=jnp.int32) % (MP - M)).astype(jnp.int32)
    src_all = jnp.concatenate([v2e_src, pad_src])
    dst_all = jnp.concatenate([v2e_dst, pad_dst])
    src_t = src_all.reshape(NW, PC, CHUNK)
    dst_t = dst_all.reshape(NW, PC, CHUNK)
    src_s = jnp.swapaxes(src_all.reshape(NW, CHUNK, PC), 1, 2)
    dst_s = jnp.swapaxes(dst_all.reshape(NW, CHUNK, PC), 1, 2)

    s2 = jnp.pad(S_features, ((0, MP - M), (0, DOUT - DS)))
    w1 = W_e2v[:DOUT]
    w2 = jnp.pad(W_e2v[DOUT:], ((0, DOUT - DS), (0, 0)))
    bx2 = b_x[None, :]
    bv2 = b_vertex[None, :]
    be2 = b_e2v[None, :]

    x_init, g, esv2 = _tc1(x_pad, W_x, W_vertex, W_att, bx2, bv2)
    esv = esv2.reshape(NP)

    num_p, den_p, cnt_p = _sc1(g, esv, src_t, dst_t)
    y = _tc2(num_p, den_p, s2, w1, w2, be2)
    xs_p = _sc2(y, src_s.reshape(NW, PC2, CH2), dst_s.reshape(NW, PC2, CH2))
    out = _tc3(xs_p, cnt_p, x_init)
    return out[:N]


# speedup vs baseline: 1.8719x; 1.2091x over previous
import functools

import jax
import jax.numpy as jnp
from jax import lax
from jax.experimental import pallas as pl
from jax.experimental.pallas import tpu as pltpu
from jax.experimental.pallas import tpu_sc as plsc

N = 10000
M = 5000
NNZ = 320000
DIN = 128
DOUT = 64
DS = 10
NEG_SLOPE = 0.2

NC = 2
NS = 16
NW = NC * NS
LANES = 16

NP = 10240
MP = 5120
NPS = NP // NS
MPS = MP // NS

CHUNK = 128
PC = 80
PT = PC * CHUNK
NNZP = NW * PT

CH2 = 256
PC2 = PT // CH2

BLK = 1024
NBN = NP // BLK
BLK2 = 512
NBM = MP // BLK2


def _elu(x):
    return jnp.where(x > 0, x, jnp.exp(jnp.minimum(x, 0.0)) - 1.0)


def _tc1_body(x_ref, wx_ref, wv_ref, wa_ref, bx_ref, bv_ref,
              xi_ref, g_ref, esv_ref, xi_s, xf_s, sv_s, gmax_s):
    p = pl.program_id(0)
    b = pl.program_id(1)

    @pl.when(p == 0)
    def _():
        x = x_ref[...]
        xf = jnp.dot(x, wv_ref[...], preferred_element_type=jnp.float32) + bv_ref[...]
        xi = jnp.dot(x, wx_ref[...], preferred_element_type=jnp.float32) + bx_ref[...]
        xi_ref[...] = xi
        xi_s[pl.ds(b * BLK, BLK), :] = xi
        xf_s[pl.ds(b * BLK, BLK), :] = xf
        sv = jnp.dot(xf, wa_ref[...], preferred_element_type=jnp.float32)
        sv = jnp.where(sv > 0, sv, NEG_SLOPE * sv)
        sv_s[pl.ds(b * BLK, BLK), :] = sv
        m = jnp.max(sv)

        @pl.when(b == 0)
        def _():
            gmax_s[0] = m

        @pl.when(b > 0)
        def _():
            gmax_s[0] = jnp.maximum(gmax_s[0], m)

    @pl.when(p == 1)
    def _():
        xi_ref[...] = xi_s[pl.ds(b * BLK, BLK), :]
        esv = jnp.exp(sv_s[pl.ds(b * BLK, BLK), :] - gmax_s[0])
        g_ref[...] = xf_s[pl.ds(b * BLK, BLK), :] * esv
        esv_ref[...] = esv


def _tc1(x_pad, w_x, w_vertex, w_att, bx2, bv2):
    return pl.pallas_call(
        _tc1_body,
        grid=(2, NBN),
        in_specs=[
            pl.BlockSpec((BLK, DIN), lambda p, b: ((1 - p) * b, 0)),
            pl.BlockSpec((DIN, DOUT), lambda p, b: (0, 0)),
            pl.BlockSpec((DIN, DOUT), lambda p, b: (0, 0)),
            pl.BlockSpec((DOUT, 1), lambda p, b: (0, 0)),
            pl.BlockSpec((1, DOUT), lambda p, b: (0, 0)),
            pl.BlockSpec((1, DOUT), lambda p, b: (0, 0)),
        ],
        out_specs=[
            pl.BlockSpec((BLK, DOUT), lambda p, b: (b, 0)),
            pl.BlockSpec((BLK, DOUT), lambda p, b: (b, 0)),
            pl.BlockSpec((BLK, 1), lambda p, b: (b, 0)),
        ],
        out_shape=[
            jax.ShapeDtypeStruct((NP, DOUT), jnp.float32),
            jax.ShapeDtypeStruct((NP, DOUT), jnp.float32),
            jax.ShapeDtypeStruct((NP, 1), jnp.float32),
        ],
        scratch_shapes=[
            pltpu.VMEM((NP, DOUT), jnp.float32),
            pltpu.VMEM((NP, DOUT), jnp.float32),
            pltpu.VMEM((NP, 1), jnp.float32),
            pltpu.SMEM((1,), jnp.float32),
        ],
    )(x_pad, w_x, w_vertex, w_att, bx2, bv2)


def _tc2_body(np_ref, dp_ref, s2_ref, w1_ref, w2_ref, be_ref, y_ref):
    num = np_ref[0] + np_ref[1]
    den = jnp.maximum(dp_ref[0] + dp_ref[1], 1e-12)
    yv = _elu(num / den[:, None])
    y_ref[...] = (
        jnp.dot(yv, w1_ref[...], preferred_element_type=jnp.float32)
        + jnp.dot(s2_ref[...], w2_ref[...], preferred_element_type=jnp.float32)
        + be_ref[...]
    )


def _tc2(num_p, den_p, s2, w1, w2, be2):
    return pl.pallas_call(
        _tc2_body,
        grid=(NBM,),
        in_specs=[
            pl.BlockSpec((NC, BLK2, DOUT), lambda b: (0, b, 0)),
            pl.BlockSpec((NC, BLK2), lambda b: (0, b)),
            pl.BlockSpec((BLK2, DOUT), lambda b: (b, 0)),
            pl.BlockSpec((DOUT, DOUT), lambda b: (0, 0)),
            pl.BlockSpec((DOUT, DOUT), lambda b: (0, 0)),
            pl.BlockSpec((1, DOUT), lambda b: (0, 0)),
        ],
        out_specs=pl.BlockSpec((BLK2, DOUT), lambda b: (b, 0)),
        out_shape=jax.ShapeDtypeStruct((MP, DOUT), jnp.float32),
    )(num_p, den_p, s2, w1, w2, be2)


def _tc3_body(xp_ref, cp_ref, xi_ref, out_ref):
    xs = xp_ref[0] + xp_ref[1]
    cnt = jnp.maximum(cp_ref[0] + cp_ref[1], 1.0)
    out_ref[...] = _elu(xs / cnt[:, None]) + xi_ref[...]


def _tc3(xs_p, cnt_p, x_init):
    return pl.pallas_call(
        _tc3_body,
        grid=(NBN,),
        in_specs=[
            pl.BlockSpec((NC, BLK, DOUT), lambda b: (0, b, 0)),
            pl.BlockSpec((NC, BLK), lambda b: (0, b)),
            pl.BlockSpec((BLK, DOUT), lambda b: (b, 0)),
        ],
        out_specs=pl.BlockSpec((BLK, DOUT), lambda b: (b, 0)),
        out_shape=jax.ShapeDtypeStruct((NP, DOUT), jnp.float32),
    )(xs_p, cnt_p, x_init)


_MESH = plsc.VectorSubcoreMesh(core_axis_name="c", subcore_axis_name="s")

_Z16 = functools.partial(jnp.zeros, (LANES,), jnp.float32)


def _zero_1d(ref, n):
    def body(i, _):
        ref[pl.ds(i * LANES, LANES)] = _Z16()
        return 0
    lax.fori_loop(0, n // LANES, body, 0)


def _zero_rows(ref, rows):
    def body(i, _):
        for k in range(DOUT // LANES):
            ref[i, pl.ds(k * LANES, LANES)] = _Z16()
        return 0
    lax.fori_loop(0, rows, body, 0)


@functools.partial(
    pl.kernel,
    out_type=[
        jax.ShapeDtypeStruct((NC, MP, DOUT), jnp.float32),
        jax.ShapeDtypeStruct((NC, MP), jnp.float32),
        jax.ShapeDtypeStruct((NC, NP), jnp.float32),
    ],
    mesh=_MESH,
    scratch_types=[
        pltpu.VMEM((PC, CHUNK), jnp.int32),
        pltpu.VMEM((PC, CHUNK), jnp.int32),
        pltpu.VMEM((NP,), jnp.float32),
        pltpu.VMEM((4, CHUNK, DOUT), jnp.float32),
        pltpu.VMEM((MP,), jnp.float32),
        pltpu.VMEM((NP,), jnp.float32),
        pltpu.VMEM((NS, MPS), jnp.float32),
        pltpu.VMEM((NS, NPS), jnp.float32),
        pltpu.VMEM((MPS,), jnp.float32),
        pltpu.VMEM((NPS,), jnp.float32),
        pltpu.VMEM_SHARED((MP, DOUT), jnp.float32),
        pltpu.VMEM_SHARED((NS, MP), jnp.float32),
        pltpu.VMEM_SHARED((NS, NP), jnp.float32),
        pltpu.SemaphoreType.DMA,
        pltpu.SemaphoreType.DMA,
        pltpu.SemaphoreType.DMA,
        pltpu.SemaphoreType.DMA,
        pltpu.SemaphoreType.DMA,
        pltpu.SemaphoreType.DMA,
        pltpu.SemaphoreType.DMA,
        pltpu.SemaphoreType.DMA,
    ],
    compiler_params=pltpu.CompilerParams(use_tc_tiling_on_sc=False, needs_layout_passes=False),
    name="sc1_v2e",
)
def _sc1(g_hbm, esv_hbm, src_hbm, dst_hbm, num_out, den_out, cnt_out,
         src_v, dst_v, esv_v, rowbuf, den_loc, cnt_loc,
         denred, cntred, denacc, cntacc, num_sh, den_sh, cnt_sh,
         gs0, gs1, gs2, gs3, ss0, ss1, ss2, ss3):
    cid = lax.axis_index("c")
    sid = lax.axis_index("s")
    wid = cid * NS + sid

    pltpu.sync_copy(src_hbm.at[wid], src_v)
    pltpu.sync_copy(dst_hbm.at[wid], dst_v)
    pltpu.sync_copy(esv_hbm, esv_v)

    _zero_rows(rowbuf.at[0], CHUNK)
    _zero_rows(rowbuf.at[1], CHUNK)
    _zero_1d(den_loc, MP)
    _zero_1d(cnt_loc, NP)
    pltpu.sync_copy(rowbuf.at[0], num_sh.at[pl.ds(sid * MPS, CHUNK)])
    pltpu.sync_copy(rowbuf.at[1], num_sh.at[pl.ds(sid * MPS + CHUNK, CHUNK)])
    pltpu.sync_copy(rowbuf.at[0, pl.ds(0, MPS - 2 * CHUNK)],
                    num_sh.at[pl.ds(sid * MPS + 2 * CHUNK, MPS - 2 * CHUNK)])
    plsc.subcore_barrier()

    ones16 = jnp.ones((LANES,), jnp.float32)
    gsems = (gs0, gs1, gs2, gs3)
    ssems = (ss0, ss1, ss2, ss3)
    KR = 4

    for b in range(KR - 1):
        pltpu.async_copy(g_hbm.at[src_v.at[b]], rowbuf.at[b], gsems[b])

    def chunk_work(jh, b):
        jj = jh * KR + b
        pb = (b - 1) % KR

        def wait_prev_scatter():
            pltpu.make_async_copy(rowbuf.at[pb],
                                  num_sh.at[dst_v.at[0]], ssems[pb]).wait()
        if b == 0:
            @pl.when(jh > 0)
            def _():
                wait_prev_scatter()
        else:
            wait_prev_scatter()

        @pl.when(jj + (KR - 1) < PC)
        def _():
            pltpu.async_copy(g_hbm.at[src_v.at[jj + KR - 1]],
                             rowbuf.at[pb], gsems[pb])

        pltpu.make_async_copy(g_hbm.at[src_v.at[jj]], rowbuf.at[b],
                              gsems[b]).wait()
        pltpu.async_copy(rowbuf.at[b], num_sh.at[dst_v.at[jj]], ssems[b],
                         add=True)
        for k in range(CHUNK // LANES):
            sidx = src_v[jj, pl.ds(k * LANES, LANES)]
            didx = dst_v[jj, pl.ds(k * LANES, LANES)]
            e = plsc.load_gather(esv_v, [sidx])
            plsc.addupdate_scatter(den_loc, [didx], e)
            plsc.addupdate_scatter(cnt_loc, [sidx], ones16)

    def body(jh, _):
        for b in range(KR):
            chunk_work(jh, b)
        return 0

    lax.fori_loop(0, PC // KR, body, 0)
    pltpu.make_async_copy(rowbuf.at[KR - 1], num_sh.at[dst_v.at[0]],
                          ssems[KR - 1]).wait()

    plsc.subcore_barrier()
    pltpu.sync_copy(den_loc, den_sh.at[sid])
    pltpu.sync_copy(cnt_loc, cnt_sh.at[sid])
    plsc.subcore_barrier()

    pltpu.sync_copy(num_sh.at[pl.ds(sid * MPS, MPS)],
                    num_out.at[cid, pl.ds(sid * MPS, MPS)])

    for r in range(NS):
        pltpu.sync_copy(den_sh.at[r, pl.ds(sid * MPS, MPS)], denred.at[r])

    def dred(i, _):
        acc = _Z16()
        for r in range(NS):
            acc = acc + denred[r, pl.ds(i * LANES, LANES)]
        denacc[pl.ds(i * LANES, LANES)] = acc
        return 0

    lax.fori_loop(0, MPS // LANES, dred, 0)
    pltpu.sync_copy(denacc, den_out.at[cid, pl.ds(sid * MPS, MPS)])

    for r in range(NS):
        pltpu.sync_copy(cnt_sh.at[r, pl.ds(sid * NPS, NPS)], cntred.at[r])

    def cred(i, _):
        acc = _Z16()
        for r in range(NS):
            acc = acc + cntred[r, pl.ds(i * LANES, LANES)]
        cntacc[pl.ds(i * LANES, LANES)] = acc
        return 0

    lax.fori_loop(0, NPS // LANES, cred, 0)
    pltpu.sync_copy(cntacc, cnt_out.at[cid, pl.ds(sid * NPS, NPS)])


@functools.partial(
    pl.kernel,
    out_type=jax.ShapeDtypeStruct((NC, NP, DOUT), jnp.float32),
    mesh=_MESH,
    scratch_types=[
        pltpu.VMEM((PC2, CH2), jnp.int32),
        pltpu.VMEM((PC2, CH2), jnp.int32),
        pltpu.VMEM((4, CH2, DOUT), jnp.float32),
        pltpu.VMEM_SHARED((NP, DOUT), jnp.float32),
        pltpu.SemaphoreType.DMA,
        pltpu.SemaphoreType.DMA,
        pltpu.SemaphoreType.DMA,
        pltpu.SemaphoreType.DMA,
        pltpu.SemaphoreType.DMA,
        pltpu.SemaphoreType.DMA,
        pltpu.SemaphoreType.DMA,
        pltpu.SemaphoreType.DMA,
    ],
    compiler_params=pltpu.CompilerParams(use_tc_tiling_on_sc=False, needs_layout_passes=False),
    name="sc2_e2v",
)
def _sc2(y_hbm, src_hbm, dst_hbm, xs_out,
         src_v, dst_v, rowbuf, xacc,
         gs0, gs1, gs2, gs3, ss0, ss1, ss2, ss3):
    cid = lax.axis_index("c")
    sid = lax.axis_index("s")
    wid = cid * NS + sid

    pltpu.sync_copy(src_hbm.at[wid], src_v)
    pltpu.sync_copy(dst_hbm.at[wid], dst_v)

    for b in range(3):
        _zero_rows(rowbuf.at[b], CH2)
    pltpu.sync_copy(rowbuf.at[0], xacc.at[pl.ds(sid * NPS, CH2)])
    pltpu.sync_copy(rowbuf.at[1], xacc.at[pl.ds(sid * NPS + CH2, CH2)])
    pltpu.sync_copy(rowbuf.at[2, pl.ds(0, NPS - 2 * CH2)],
                    xacc.at[pl.ds(sid * NPS + 2 * CH2, NPS - 2 * CH2)])
    plsc.subcore_barrier()

    gsems = (gs0, gs1, gs2, gs3)
    ssems = (ss0, ss1, ss2, ss3)
    KR = 4

    for b in range(KR - 1):
        pltpu.async_copy(y_hbm.at[dst_v.at[b]], rowbuf.at[b], gsems[b])

    def chunk_work(jh, b):
        jj = jh * KR + b
        pb = (b - 1) % KR

        def wait_prev_scatter():
            pltpu.make_async_copy(rowbuf.at[pb],
                                  xacc.at[src_v.at[0]], ssems[pb]).wait()
        if b == 0:
            @pl.when(jh > 0)
            def _():
                wait_prev_scatter()
        else:
            wait_prev_scatter()

        @pl.when(jj + (KR - 1) < PC2)
        def _():
            pltpu.async_copy(y_hbm.at[dst_v.at[jj + KR - 1]],
                             rowbuf.at[pb], gsems[pb])

        pltpu.make_async_copy(y_hbm.at[dst_v.at[jj]], rowbuf.at[b],
                              gsems[b]).wait()
        pltpu.async_copy(rowbuf.at[b], xacc.at[src_v.at[jj]], ssems[b],
                         add=True)

    def body(jh, _):
        for b in range(KR):
            chunk_work(jh, b)
        return 0

    lax.fori_loop(0, PC2 // KR, body, 0)
    pltpu.make_async_copy(rowbuf.at[KR - 1], xacc.at[src_v.at[0]],
                          ssems[KR - 1]).wait()

    plsc.subcore_barrier()
    pltpu.sync_copy(xacc.at[pl.ds(sid * NPS, NPS)],
                    xs_out.at[cid, pl.ds(sid * NPS, NPS)])


def kernel(X, v2e_src, v2e_dst, S_features, W_x, b_x, W_vertex, b_vertex,
           W_group, b_group, W_att, W_e2v, b_e2v):
    x_pad = jnp.pad(X, ((0, NP - N), (0, 0)))
    npad = NNZP - NNZ
    pad_src = (N + jnp.arange(npad, dtype=jnp.int32) % (NP - N)).astype(jnp.int32)
    pad_dst = (M + jnp.arange(npad, dtype=jnp.int32) % (MP - M)).astype(jnp.int32)
    src_all = jnp.concatenate([v2e_src, pad_src])
    dst_all = jnp.concatenate([v2e_dst, pad_dst])
    src_t = src_all.reshape(NW, PC, CHUNK)
    dst_t = dst_all.reshape(NW, PC, CHUNK)
    src_s = jnp.swapaxes(src_all.reshape(NW, CHUNK, PC), 1, 2)
    dst_s = jnp.swapaxes(dst_all.reshape(NW, CHUNK, PC), 1, 2)

    s2 = jnp.pad(S_features, ((0, MP - M), (0, DOUT - DS)))
    w1 = W_e2v[:DOUT]
    w2 = jnp.pad(W_e2v[DOUT:], ((0, DOUT - DS), (0, 0)))
    bx2 = b_x[None, :]
    bv2 = b_vertex[None, :]
    be2 = b_e2v[None, :]

    x_init, g, esv2 = _tc1(x_pad, W_x, W_vertex, W_att, bx2, bv2)
    esv = esv2.reshape(NP)

    num_p, den_p, cnt_p = _sc1(g, esv, src_t, dst_t)
    y = _tc2(num_p, den_p, s2, w1, w2, be2)
    xs_p = _sc2(y, src_s.reshape(NW, PC2, CH2), dst_s.reshape(NW, PC2, CH2))
    out = _tc3(xs_p, cnt_p, x_init)
    return out[:N]


# speedup vs baseline: 1.9469x; 1.0401x over previous
import functools

import jax
import jax.numpy as jnp
from jax import lax
from jax.experimental import pallas as pl
from jax.experimental.pallas import tpu as pltpu
from jax.experimental.pallas import tpu_sc as plsc

N = 10000
M = 5000
NNZ = 320000
DIN = 128
DOUT = 64
DS = 10
NEG_SLOPE = 0.2

NC = 2
NS = 16
NW = NC * NS
LANES = 16

NP = 10240
MP = 5120
NPS = NP // NS
MPS = MP // NS

CHUNK = 128
PC = 80
PT = PC * CHUNK
NNZP = NW * PT

CH2 = 256
PC2 = PT // CH2

BLK = 2048
NBN = NP // BLK
BLK2 = 1024
NBM = MP // BLK2


def _elu(x):
    return jnp.where(x > 0, x, jnp.exp(jnp.minimum(x, 0.0)) - 1.0)


def _tc1_body(x_ref, wx_ref, wv_ref, wa_ref, bx_ref, bv_ref,
              xi_ref, g_ref, esv_ref, xi_s, xf_s, sv_s, gmax_s):
    p = pl.program_id(0)
    b = pl.program_id(1)

    @pl.when(p == 0)
    def _():
        x = x_ref[...]
        xf = jnp.dot(x, wv_ref[...], preferred_element_type=jnp.float32) + bv_ref[...]
        xi = jnp.dot(x, wx_ref[...], preferred_element_type=jnp.float32) + bx_ref[...]
        xi_ref[...] = xi
        xi_s[pl.ds(b * BLK, BLK), :] = xi
        xf_s[pl.ds(b * BLK, BLK), :] = xf
        sv = jnp.dot(xf, wa_ref[...], preferred_element_type=jnp.float32)
        sv = jnp.where(sv > 0, sv, NEG_SLOPE * sv)
        sv_s[pl.ds(b * BLK, BLK), :] = sv
        m = jnp.max(sv)

        @pl.when(b == 0)
        def _():
            gmax_s[0] = m

        @pl.when(b > 0)
        def _():
            gmax_s[0] = jnp.maximum(gmax_s[0], m)

    @pl.when(p == 1)
    def _():
        xi_ref[...] = xi_s[pl.ds(b * BLK, BLK), :]
        esv = jnp.exp(sv_s[pl.ds(b * BLK, BLK), :] - gmax_s[0])
        g_ref[...] = xf_s[pl.ds(b * BLK, BLK), :] * esv
        esv_ref[...] = esv


def _tc1(x_pad, w_x, w_vertex, w_att, bx2, bv2):
    return pl.pallas_call(
        _tc1_body,
        grid=(2, NBN),
        in_specs=[
            pl.BlockSpec((BLK, DIN), lambda p, b: ((1 - p) * b, 0)),
            pl.BlockSpec((DIN, DOUT), lambda p, b: (0, 0)),
            pl.BlockSpec((DIN, DOUT), lambda p, b: (0, 0)),
            pl.BlockSpec((DOUT, 1), lambda p, b: (0, 0)),
            pl.BlockSpec((1, DOUT), lambda p, b: (0, 0)),
            pl.BlockSpec((1, DOUT), lambda p, b: (0, 0)),
        ],
        out_specs=[
            pl.BlockSpec((BLK, DOUT), lambda p, b: (b, 0)),
            pl.BlockSpec((BLK, DOUT), lambda p, b: (b, 0)),
            pl.BlockSpec((BLK, 1), lambda p, b: (b, 0)),
        ],
        out_shape=[
            jax.ShapeDtypeStruct((NP, DOUT), jnp.float32),
            jax.ShapeDtypeStruct((NP, DOUT), jnp.float32),
            jax.ShapeDtypeStruct((NP, 1), jnp.float32),
        ],
        scratch_shapes=[
            pltpu.VMEM((NP, DOUT), jnp.float32),
            pltpu.VMEM((NP, DOUT), jnp.float32),
            pltpu.VMEM((NP, 1), jnp.float32),
            pltpu.SMEM((1,), jnp.float32),
        ],
    )(x_pad, w_x, w_vertex, w_att, bx2, bv2)


def _tc2_body(np_ref, dp_ref, s2_ref, w1_ref, w2_ref, be_ref, y_ref):
    num = np_ref[0] + np_ref[1]
    den = jnp.maximum(dp_ref[0] + dp_ref[1], 1e-12)
    yv = _elu(num / den[:, None])
    y_ref[...] = (
        jnp.dot(yv, w1_ref[...], preferred_element_type=jnp.float32)
        + jnp.dot(s2_ref[...], w2_ref[...], preferred_element_type=jnp.float32)
        + be_ref[...]
    )


def _tc2(num_p, den_p, s2, w1, w2, be2):
    return pl.pallas_call(
        _tc2_body,
        grid=(NBM,),
        in_specs=[
            pl.BlockSpec((NC, BLK2, DOUT), lambda b: (0, b, 0)),
            pl.BlockSpec((NC, BLK2), lambda b: (0, b)),
            pl.BlockSpec((BLK2, DOUT), lambda b: (b, 0)),
            pl.BlockSpec((DOUT, DOUT), lambda b: (0, 0)),
            pl.BlockSpec((DOUT, DOUT), lambda b: (0, 0)),
            pl.BlockSpec((1, DOUT), lambda b: (0, 0)),
        ],
        out_specs=pl.BlockSpec((BLK2, DOUT), lambda b: (b, 0)),
        out_shape=jax.ShapeDtypeStruct((MP, DOUT), jnp.float32),
    )(num_p, den_p, s2, w1, w2, be2)


def _tc3_body(xp_ref, cp_ref, xi_ref, out_ref):
    xs = xp_ref[0] + xp_ref[1]
    cnt = jnp.maximum(cp_ref[0] + cp_ref[1], 1.0)
    out_ref[...] = _elu(xs / cnt[:, None]) + xi_ref[...]


def _tc3(xs_p, cnt_p, x_init):
    return pl.pallas_call(
        _tc3_body,
        grid=(NBN,),
        in_specs=[
            pl.BlockSpec((NC, BLK, DOUT), lambda b: (0, b, 0)),
            pl.BlockSpec((NC, BLK), lambda b: (0, b)),
            pl.BlockSpec((BLK, DOUT), lambda b: (b, 0)),
        ],
        out_specs=pl.BlockSpec((BLK, DOUT), lambda b: (b, 0)),
        out_shape=jax.ShapeDtypeStruct((N, DOUT), jnp.float32),
    )(xs_p, cnt_p, x_init)


_MESH = plsc.VectorSubcoreMesh(core_axis_name="c", subcore_axis_name="s")

_Z16 = functools.partial(jnp.zeros, (LANES,), jnp.float32)


def _zero_1d(ref, n):
    def body(i, _):
        ref[pl.ds(i * LANES, LANES)] = _Z16()
        return 0
    lax.fori_loop(0, n // LANES, body, 0)


def _zero_rows(ref, rows):
    def body(i, _):
        for k in range(DOUT // LANES):
            ref[i, pl.ds(k * LANES, LANES)] = _Z16()
        return 0
    lax.fori_loop(0, rows, body, 0)


@functools.partial(
    pl.kernel,
    out_type=[
        jax.ShapeDtypeStruct((NC, MP, DOUT), jnp.float32),
        jax.ShapeDtypeStruct((NC, MP), jnp.float32),
        jax.ShapeDtypeStruct((NC, NP), jnp.float32),
    ],
    mesh=_MESH,
    scratch_types=[
        pltpu.VMEM((PC, CHUNK), jnp.int32),
        pltpu.VMEM((PC, CHUNK), jnp.int32),
        pltpu.VMEM((NP,), jnp.float32),
        pltpu.VMEM((4, CHUNK, DOUT), jnp.float32),
        pltpu.VMEM((MP,), jnp.float32),
        pltpu.VMEM((NP,), jnp.float32),
        pltpu.VMEM((NS, MPS), jnp.float32),
        pltpu.VMEM((NS, NPS), jnp.float32),
        pltpu.VMEM((MPS,), jnp.float32),
        pltpu.VMEM((NPS,), jnp.float32),
        pltpu.VMEM_SHARED((MP, DOUT), jnp.float32),
        pltpu.VMEM_SHARED((NS, MP), jnp.float32),
        pltpu.VMEM_SHARED((NS, NP), jnp.float32),
        pltpu.SemaphoreType.DMA,
        pltpu.SemaphoreType.DMA,
        pltpu.SemaphoreType.DMA,
        pltpu.SemaphoreType.DMA,
        pltpu.SemaphoreType.DMA,
        pltpu.SemaphoreType.DMA,
        pltpu.SemaphoreType.DMA,
        pltpu.SemaphoreType.DMA,
    ],
    compiler_params=pltpu.CompilerParams(use_tc_tiling_on_sc=False, needs_layout_passes=False),
    name="sc1_v2e",
)
def _sc1(g_hbm, esv_hbm, src_hbm, dst_hbm, num_out, den_out, cnt_out,
         src_v, dst_v, esv_v, rowbuf, den_loc, cnt_loc,
         denred, cntred, denacc, cntacc, num_sh, den_sh, cnt_sh,
         gs0, gs1, gs2, gs3, ss0, ss1, ss2, ss3):
    cid = lax.axis_index("c")
    sid = lax.axis_index("s")
    wid = cid * NS + sid

    pltpu.sync_copy(src_hbm.at[wid], src_v)
    pltpu.sync_copy(dst_hbm.at[wid], dst_v)
    pltpu.sync_copy(esv_hbm, esv_v)

    _zero_rows(rowbuf.at[0], CHUNK)
    _zero_rows(rowbuf.at[1], CHUNK)
    _zero_1d(den_loc, MP)
    _zero_1d(cnt_loc, NP)
    pltpu.sync_copy(rowbuf.at[0], num_sh.at[pl.ds(sid * MPS, CHUNK)])
    pltpu.sync_copy(rowbuf.at[1], num_sh.at[pl.ds(sid * MPS + CHUNK, CHUNK)])
    pltpu.sync_copy(rowbuf.at[0, pl.ds(0, MPS - 2 * CHUNK)],
                    num_sh.at[pl.ds(sid * MPS + 2 * CHUNK, MPS - 2 * CHUNK)])
    plsc.subcore_barrier()

    ones16 = jnp.ones((LANES,), jnp.float32)
    gsems = (gs0, gs1, gs2, gs3)
    ssems = (ss0, ss1, ss2, ss3)
    KR = 4

    for b in range(KR - 1):
        pltpu.async_copy(g_hbm.at[src_v.at[b]], rowbuf.at[b], gsems[b])

    def chunk_work(jh, b):
        jj = jh * KR + b
        pb = (b - 1) % KR

        def wait_prev_scatter():
            pltpu.make_async_copy(rowbuf.at[pb],
                                  num_sh.at[dst_v.at[0]], ssems[pb]).wait()
        if b == 0:
            @pl.when(jh > 0)
            def _():
                wait_prev_scatter()
        else:
            wait_prev_scatter()

        @pl.when(jj + (KR - 1) < PC)
        def _():
            pltpu.async_copy(g_hbm.at[src_v.at[jj + KR - 1]],
                             rowbuf.at[pb], gsems[pb])

        pltpu.make_async_copy(g_hbm.at[src_v.at[jj]], rowbuf.at[b],
                              gsems[b]).wait()
        pltpu.async_copy(rowbuf.at[b], num_sh.at[dst_v.at[jj]], ssems[b],
                         add=True)
        for k in range(CHUNK // LANES):
            sidx = src_v[jj, pl.ds(k * LANES, LANES)]
            didx = dst_v[jj, pl.ds(k * LANES, LANES)]
            e = plsc.load_gather(esv_v, [sidx])
            plsc.addupdate_scatter(den_loc, [didx], e)
            plsc.addupdate_scatter(cnt_loc, [sidx], ones16)

    def body(jh, _):
        for b in range(KR):
            chunk_work(jh, b)
        return 0

    lax.fori_loop(0, PC // KR, body, 0)
    pltpu.make_async_copy(rowbuf.at[KR - 1], num_sh.at[dst_v.at[0]],
                          ssems[KR - 1]).wait()

    plsc.subcore_barrier()
    pltpu.sync_copy(den_loc, den_sh.at[sid])
    pltpu.sync_copy(cnt_loc, cnt_sh.at[sid])
    plsc.subcore_barrier()

    pltpu.sync_copy(num_sh.at[pl.ds(sid * MPS, MPS)],
                    num_out.at[cid, pl.ds(sid * MPS, MPS)])

    for r in range(NS):
        pltpu.sync_copy(den_sh.at[r, pl.ds(sid * MPS, MPS)], denred.at[r])

    def dred(i, _):
        acc = _Z16()
        for r in range(NS):
            acc = acc + denred[r, pl.ds(i * LANES, LANES)]
        denacc[pl.ds(i * LANES, LANES)] = acc
        return 0

    lax.fori_loop(0, MPS // LANES, dred, 0)
    pltpu.sync_copy(denacc, den_out.at[cid, pl.ds(sid * MPS, MPS)])

    for r in range(NS):
        pltpu.sync_copy(cnt_sh.at[r, pl.ds(sid * NPS, NPS)], cntred.at[r])

    def cred(i, _):
        acc = _Z16()
        for r in range(NS):
            acc = acc + cntred[r, pl.ds(i * LANES, LANES)]
        cntacc[pl.ds(i * LANES, LANES)] = acc
        return 0

    lax.fori_loop(0, NPS // LANES, cred, 0)
    pltpu.sync_copy(cntacc, cnt_out.at[cid, pl.ds(sid * NPS, NPS)])


@functools.partial(
    pl.kernel,
    out_type=jax.ShapeDtypeStruct((NC, NP, DOUT), jnp.float32),
    mesh=_MESH,
    scratch_types=[
        pltpu.VMEM((PC2, CH2), jnp.int32),
        pltpu.VMEM((PC2, CH2), jnp.int32),
        pltpu.VMEM((4, CH2, DOUT), jnp.float32),
        pltpu.VMEM_SHARED((NP, DOUT), jnp.float32),
        pltpu.SemaphoreType.DMA,
        pltpu.SemaphoreType.DMA,
        pltpu.SemaphoreType.DMA,
        pltpu.SemaphoreType.DMA,
        pltpu.SemaphoreType.DMA,
        pltpu.SemaphoreType.DMA,
        pltpu.SemaphoreType.DMA,
        pltpu.SemaphoreType.DMA,
    ],
    compiler_params=pltpu.CompilerParams(use_tc_tiling_on_sc=False, needs_layout_passes=False),
    name="sc2_e2v",
)
def _sc2(y_hbm, src_hbm, dst_hbm, xs_out,
         src_v, dst_v, rowbuf, xacc,
         gs0, gs1, gs2, gs3, ss0, ss1, ss2, ss3):
    cid = lax.axis_index("c")
    sid = lax.axis_index("s")
    wid = cid * NS + sid

    pltpu.sync_copy(src_hbm.at[wid], src_v)
    pltpu.sync_copy(dst_hbm.at[wid], dst_v)

    for b in range(3):
        _zero_rows(rowbuf.at[b], CH2)
    pltpu.sync_copy(rowbuf.at[0], xacc.at[pl.ds(sid * NPS, CH2)])
    pltpu.sync_copy(rowbuf.at[1], xacc.at[pl.ds(sid * NPS + CH2, CH2)])
    pltpu.sync_copy(rowbuf.at[2, pl.ds(0, NPS - 2 * CH2)],
                    xacc.at[pl.ds(sid * NPS + 2 * CH2, NPS - 2 * CH2)])
    plsc.subcore_barrier()

    gsems = (gs0, gs1, gs2, gs3)
    ssems = (ss0, ss1, ss2, ss3)
    KR = 4

    for b in range(KR - 1):
        pltpu.async_copy(y_hbm.at[dst_v.at[b]], rowbuf.at[b], gsems[b])

    def chunk_work(jh, b):
        jj = jh * KR + b
        pb = (b - 1) % KR

        def wait_prev_scatter():
            pltpu.make_async_copy(rowbuf.at[pb],
                                  xacc.at[src_v.at[0]], ssems[pb]).wait()
        if b == 0:
            @pl.when(jh > 0)
            def _():
                wait_prev_scatter()
        else:
            wait_prev_scatter()

        @pl.when(jj + (KR - 1) < PC2)
        def _():
            pltpu.async_copy(y_hbm.at[dst_v.at[jj + KR - 1]],
                             rowbuf.at[pb], gsems[pb])

        pltpu.make_async_copy(y_hbm.at[dst_v.at[jj]], rowbuf.at[b],
                              gsems[b]).wait()
        pltpu.async_copy(rowbuf.at[b], xacc.at[src_v.at[jj]], ssems[b],
                         add=True)

    def body(jh, _):
        for b in range(KR):
            chunk_work(jh, b)
        return 0

    lax.fori_loop(0, PC2 // KR, body, 0)
    pltpu.make_async_copy(rowbuf.at[KR - 1], xacc.at[src_v.at[0]],
                          ssems[KR - 1]).wait()

    plsc.subcore_barrier()
    pltpu.sync_copy(xacc.at[pl.ds(sid * NPS, NPS)],
                    xs_out.at[cid, pl.ds(sid * NPS, NPS)])


def kernel(X, v2e_src, v2e_dst, S_features, W_x, b_x, W_vertex, b_vertex,
           W_group, b_group, W_att, W_e2v, b_e2v):
    x_pad = jnp.pad(X, ((0, NP - N), (0, 0)))
    npad = NNZP - NNZ
    pad_src = (N + jnp.arange(npad, dtype=jnp.int32) % (NP - N)).astype(jnp.int32)
    pad_dst = (M + jnp.arange(npad, dtype=jnp.int32) % (MP - M)).astype(jnp.int32)
    src_all = jnp.concatenate([v2e_src, pad_src])
    dst_all = jnp.concatenate([v2e_dst, pad_dst])
    src_t = src_all.reshape(NW, PC, CHUNK)
    dst_t = dst_all.reshape(NW, PC, CHUNK)
    src_s = jnp.swapaxes(src_all.reshape(NW, CHUNK, PC), 1, 2)
    dst_s = jnp.swapaxes(dst_all.reshape(NW, CHUNK, PC), 1, 2)

    s2 = jnp.pad(S_features, ((0, MP - M), (0, DOUT - DS)))
    w1 = W_e2v[:DOUT]
    w2 = jnp.pad(W_e2v[DOUT:], ((0, DOUT - DS), (0, 0)))
    bx2 = b_x[None, :]
    bv2 = b_vertex[None, :]
    be2 = b_e2v[None, :]

    x_init, g, esv2 = _tc1(x_pad, W_x, W_vertex, W_att, bx2, bv2)
    esv = esv2.reshape(NP)

    num_p, den_p, cnt_p = _sc1(g, esv, src_t, dst_t)
    y = _tc2(num_p, den_p, s2, w1, w2, be2)
    xs_p = _sc2(y, src_s.reshape(NW, PC2, CH2), dst_s.reshape(NW, PC2, CH2))
    return _tc3(xs_p, cnt_p, x_init)


# speedup vs baseline: 1.9504x; 1.0018x over previous
import functools

import jax
import jax.numpy as jnp
from jax import lax
from jax.experimental import pallas as pl
from jax.experimental.pallas import tpu as pltpu
from jax.experimental.pallas import tpu_sc as plsc

N = 10000
M = 5000
NNZ = 320000
DIN = 128
DOUT = 64
DS = 10
NEG_SLOPE = 0.2

NC = 2
NS = 16
NW = NC * NS
LANES = 16

NP = 10240
MP = 5120
NPS = NP // NS
MPS = MP // NS

CHUNK = 128
PC = 80
PT = PC * CHUNK
NNZP = NW * PT

CH2 = 256
PC2 = PT // CH2

BLK = 2048
NBN = NP // BLK
BLK2 = 1024
NBM = MP // BLK2


def _elu(x):
    return jnp.where(x > 0, x, jnp.exp(jnp.minimum(x, 0.0)) - 1.0)


def _tc1_body(x_ref, wx_ref, wv_ref, wa_ref, bx_ref, bv_ref,
              xi_ref, g_ref, esv_ref, xi_s, xf_s, sv_s, gmax_s):
    p = pl.program_id(0)
    b = pl.program_id(1)

    @pl.when(p == 0)
    def _():
        x = x_ref[...]
        xf = jnp.dot(x, wv_ref[...], preferred_element_type=jnp.float32) + bv_ref[...]
        xi = jnp.dot(x, wx_ref[...], preferred_element_type=jnp.float32) + bx_ref[...]
        xi_ref[...] = xi
        xi_s[pl.ds(b * BLK, BLK), :] = xi
        xf_s[pl.ds(b * BLK, BLK), :] = xf
        sv = jnp.dot(xf, wa_ref[...], preferred_element_type=jnp.float32)
        sv = jnp.where(sv > 0, sv, NEG_SLOPE * sv)
        sv_s[pl.ds(b * BLK, BLK), :] = sv
        m = jnp.max(sv)

        @pl.when(b == 0)
        def _():
            gmax_s[0] = m

        @pl.when(b > 0)
        def _():
            gmax_s[0] = jnp.maximum(gmax_s[0], m)

    @pl.when(p == 1)
    def _():
        xi_ref[...] = xi_s[pl.ds(b * BLK, BLK), :]
        esv = jnp.exp(sv_s[pl.ds(b * BLK, BLK), :] - gmax_s[0])
        g_ref[...] = xf_s[pl.ds(b * BLK, BLK), :] * esv
        esv_ref[...] = esv


def _tc1(x_pad, w_x, w_vertex, w_att, bx2, bv2):
    return pl.pallas_call(
        _tc1_body,
        grid=(2, NBN),
        in_specs=[
            pl.BlockSpec((BLK, DIN), lambda p, b: ((1 - p) * b, 0)),
            pl.BlockSpec((DIN, DOUT), lambda p, b: (0, 0)),
            pl.BlockSpec((DIN, DOUT), lambda p, b: (0, 0)),
            pl.BlockSpec((DOUT, 1), lambda p, b: (0, 0)),
            pl.BlockSpec((1, DOUT), lambda p, b: (0, 0)),
            pl.BlockSpec((1, DOUT), lambda p, b: (0, 0)),
        ],
        out_specs=[
            pl.BlockSpec((BLK, DOUT), lambda p, b: (b, 0)),
            pl.BlockSpec((BLK, DOUT), lambda p, b: (b, 0)),
            pl.BlockSpec((BLK, 1), lambda p, b: (b, 0)),
        ],
        out_shape=[
            jax.ShapeDtypeStruct((NP, DOUT), jnp.float32),
            jax.ShapeDtypeStruct((NP, DOUT), jnp.float32),
            jax.ShapeDtypeStruct((NP, 1), jnp.float32),
        ],
        scratch_shapes=[
            pltpu.VMEM((NP, DOUT), jnp.float32),
            pltpu.VMEM((NP, DOUT), jnp.float32),
            pltpu.VMEM((NP, 1), jnp.float32),
            pltpu.SMEM((1,), jnp.float32),
        ],
    )(x_pad, w_x, w_vertex, w_att, bx2, bv2)


def _tc2_body(np_ref, dp_ref, s2_ref, w1_ref, w2_ref, be_ref, y_ref):
    num = np_ref[0] + np_ref[1]
    den = jnp.maximum(jnp.sum(dp_ref[...], axis=(0, 1)), 1e-12)
    yv = _elu(num / den[:, None])
    y_ref[...] = (
        jnp.dot(yv, w1_ref[...], preferred_element_type=jnp.float32)
        + jnp.dot(s2_ref[...], w2_ref[...], preferred_element_type=jnp.float32)
        + be_ref[...]
    )


def _tc2(num_p, den_p, s2, w1, w2, be2):
    return pl.pallas_call(
        _tc2_body,
        grid=(NBM,),
        in_specs=[
            pl.BlockSpec((NC, BLK2, DOUT), lambda b: (0, b, 0)),
            pl.BlockSpec((NC, NS, BLK2), lambda b: (0, 0, b)),
            pl.BlockSpec((BLK2, DOUT), lambda b: (b, 0)),
            pl.BlockSpec((DOUT, DOUT), lambda b: (0, 0)),
            pl.BlockSpec((DOUT, DOUT), lambda b: (0, 0)),
            pl.BlockSpec((1, DOUT), lambda b: (0, 0)),
        ],
        out_specs=pl.BlockSpec((BLK2, DOUT), lambda b: (b, 0)),
        out_shape=jax.ShapeDtypeStruct((MP, DOUT), jnp.float32),
    )(num_p, den_p, s2, w1, w2, be2)


def _tc3_body(xp_ref, cp_ref, xi_ref, out_ref):
    xs = xp_ref[0] + xp_ref[1]
    cnt = jnp.maximum(jnp.sum(cp_ref[...], axis=(0, 1)), 1.0)
    out_ref[...] = _elu(xs / cnt[:, None]) + xi_ref[...]


def _tc3(xs_p, cnt_p, x_init):
    return pl.pallas_call(
        _tc3_body,
        grid=(NBN,),
        in_specs=[
            pl.BlockSpec((NC, BLK, DOUT), lambda b: (0, b, 0)),
            pl.BlockSpec((NC, NS, BLK), lambda b: (0, 0, b)),
            pl.BlockSpec((BLK, DOUT), lambda b: (b, 0)),
        ],
        out_specs=pl.BlockSpec((BLK, DOUT), lambda b: (b, 0)),
        out_shape=jax.ShapeDtypeStruct((N, DOUT), jnp.float32),
    )(xs_p, cnt_p, x_init)


_MESH = plsc.VectorSubcoreMesh(core_axis_name="c", subcore_axis_name="s")

_Z16 = functools.partial(jnp.zeros, (LANES,), jnp.float32)


def _zero_1d(ref, n):
    def body(i, _):
        ref[pl.ds(i * LANES, LANES)] = _Z16()
        return 0
    lax.fori_loop(0, n // LANES, body, 0)


def _zero_rows(ref, rows):
    def body(i, _):
        for k in range(DOUT // LANES):
            ref[i, pl.ds(k * LANES, LANES)] = _Z16()
        return 0
    lax.fori_loop(0, rows, body, 0)


@functools.partial(
    pl.kernel,
    out_type=[
        jax.ShapeDtypeStruct((NC, MP, DOUT), jnp.float32),
        jax.ShapeDtypeStruct((NC, NS, MP), jnp.float32),
        jax.ShapeDtypeStruct((NC, NS, NP), jnp.float32),
    ],
    mesh=_MESH,
    scratch_types=[
        pltpu.VMEM((PC2, CH2), jnp.int32),
        pltpu.VMEM((PC2, CH2), jnp.int32),
        pltpu.VMEM((NP,), jnp.float32),
        pltpu.VMEM((2, CH2, DOUT), jnp.float32),
        pltpu.VMEM((MP,), jnp.float32),
        pltpu.VMEM((NP,), jnp.float32),
        pltpu.VMEM_SHARED((MP, DOUT), jnp.float32),
        pltpu.SemaphoreType.DMA,
        pltpu.SemaphoreType.DMA,
        pltpu.SemaphoreType.DMA,
        pltpu.SemaphoreType.DMA,
    ],
    compiler_params=pltpu.CompilerParams(use_tc_tiling_on_sc=False, needs_layout_passes=False),
    name="sc1_v2e",
)
def _sc1(g_hbm, esv_hbm, src_hbm, dst_hbm, num_out, den_out, cnt_out,
         src_v, dst_v, esv_v, rowbuf, den_loc, cnt_loc, num_sh,
         gs0, gs1, ss0, ss1):
    cid = lax.axis_index("c")
    sid = lax.axis_index("s")
    wid = cid * NS + sid

    pltpu.sync_copy(src_hbm.at[wid], src_v)
    pltpu.sync_copy(dst_hbm.at[wid], dst_v)
    pltpu.sync_copy(esv_hbm, esv_v)

    _zero_rows(rowbuf.at[0], CH2)
    _zero_rows(rowbuf.at[1], MPS - CH2)
    _zero_1d(den_loc, MP)
    _zero_1d(cnt_loc, NP)
    pltpu.sync_copy(rowbuf.at[0], num_sh.at[pl.ds(sid * MPS, CH2)])
    pltpu.sync_copy(rowbuf.at[1, pl.ds(0, MPS - CH2)],
                    num_sh.at[pl.ds(sid * MPS + CH2, MPS - CH2)])
    plsc.subcore_barrier()

    ones16 = jnp.ones((LANES,), jnp.float32)
    gsems = (gs0, gs1)
    ssems = (ss0, ss1)
    KR = 2

    pltpu.async_copy(g_hbm.at[src_v.at[0]], rowbuf.at[0], gsems[0])

    def chunk_work(jh, b):
        jj = jh * KR + b
        pb = 1 - b

        def wait_prev_scatter():
            pltpu.make_async_copy(rowbuf.at[pb],
                                  num_sh.at[dst_v.at[0]], ssems[pb]).wait()
        if b == 0:
            @pl.when(jh > 0)
            def _():
                wait_prev_scatter()
        else:
            wait_prev_scatter()

        @pl.when(jj + 1 < PC2)
        def _():
            pltpu.async_copy(g_hbm.at[src_v.at[jj + 1]],
                             rowbuf.at[pb], gsems[pb])

        pltpu.make_async_copy(g_hbm.at[src_v.at[jj]], rowbuf.at[b],
                              gsems[b]).wait()
        pltpu.async_copy(rowbuf.at[b], num_sh.at[dst_v.at[jj]], ssems[b],
                         add=True)
        for k in range(CH2 // LANES):
            sidx = src_v[jj, pl.ds(k * LANES, LANES)]
            didx = dst_v[jj, pl.ds(k * LANES, LANES)]
            e = plsc.load_gather(esv_v, [sidx])
            plsc.addupdate_scatter(den_loc, [didx], e)
            plsc.addupdate_scatter(cnt_loc, [sidx], ones16)

    def body(jh, _):
        for b in range(KR):
            chunk_work(jh, b)
        return 0

    lax.fori_loop(0, PC2 // KR, body, 0)
    pltpu.make_async_copy(rowbuf.at[KR - 1], num_sh.at[dst_v.at[0]],
                          ssems[KR - 1]).wait()

    plsc.subcore_barrier()
    pltpu.sync_copy(num_sh.at[pl.ds(sid * MPS, MPS)],
                    num_out.at[cid, pl.ds(sid * MPS, MPS)])
    pltpu.sync_copy(den_loc, den_out.at[cid, sid])
    pltpu.sync_copy(cnt_loc, cnt_out.at[cid, sid])


@functools.partial(
    pl.kernel,
    out_type=jax.ShapeDtypeStruct((NC, NP, DOUT), jnp.float32),
    mesh=_MESH,
    scratch_types=[
        pltpu.VMEM((PC2, CH2), jnp.int32),
        pltpu.VMEM((PC2, CH2), jnp.int32),
        pltpu.VMEM((4, CH2, DOUT), jnp.float32),
        pltpu.VMEM_SHARED((NP, DOUT), jnp.float32),
        pltpu.SemaphoreType.DMA,
        pltpu.SemaphoreType.DMA,
        pltpu.SemaphoreType.DMA,
        pltpu.SemaphoreType.DMA,
        pltpu.SemaphoreType.DMA,
        pltpu.SemaphoreType.DMA,
        pltpu.SemaphoreType.DMA,
        pltpu.SemaphoreType.DMA,
    ],
    compiler_params=pltpu.CompilerParams(use_tc_tiling_on_sc=False, needs_layout_passes=False),
    name="sc2_e2v",
)
def _sc2(y_hbm, src_hbm, dst_hbm, xs_out,
         src_v, dst_v, rowbuf, xacc,
         gs0, gs1, gs2, gs3, ss0, ss1, ss2, ss3):
    cid = lax.axis_index("c")
    sid = lax.axis_index("s")
    wid = cid * NS + sid

    pltpu.sync_copy(src_hbm.at[wid], src_v)
    pltpu.sync_copy(dst_hbm.at[wid], dst_v)

    for b in range(3):
        _zero_rows(rowbuf.at[b], CH2)
    pltpu.sync_copy(rowbuf.at[0], xacc.at[pl.ds(sid * NPS, CH2)])
    pltpu.sync_copy(rowbuf.at[1], xacc.at[pl.ds(sid * NPS + CH2, CH2)])
    pltpu.sync_copy(rowbuf.at[2, pl.ds(0, NPS - 2 * CH2)],
                    xacc.at[pl.ds(sid * NPS + 2 * CH2, NPS - 2 * CH2)])
    plsc.subcore_barrier()

    gsems = (gs0, gs1, gs2, gs3)
    ssems = (ss0, ss1, ss2, ss3)
    KR = 4

    for b in range(KR - 1):
        pltpu.async_copy(y_hbm.at[dst_v.at[b]], rowbuf.at[b], gsems[b])

    def chunk_work(jh, b):
        jj = jh * KR + b
        pb = (b - 1) % KR

        def wait_prev_scatter():
            pltpu.make_async_copy(rowbuf.at[pb],
                                  xacc.at[src_v.at[0]], ssems[pb]).wait()
        if b == 0:
            @pl.when(jh > 0)
            def _():
                wait_prev_scatter()
        else:
            wait_prev_scatter()

        @pl.when(jj + (KR - 1) < PC2)
        def _():
            pltpu.async_copy(y_hbm.at[dst_v.at[jj + KR - 1]],
                             rowbuf.at[pb], gsems[pb])

        pltpu.make_async_copy(y_hbm.at[dst_v.at[jj]], rowbuf.at[b],
                              gsems[b]).wait()
        pltpu.async_copy(rowbuf.at[b], xacc.at[src_v.at[jj]], ssems[b],
                         add=True)

    def body(jh, _):
        for b in range(KR):
            chunk_work(jh, b)
        return 0

    lax.fori_loop(0, PC2 // KR, body, 0)
    pltpu.make_async_copy(rowbuf.at[KR - 1], xacc.at[src_v.at[0]],
                          ssems[KR - 1]).wait()

    plsc.subcore_barrier()
    pltpu.sync_copy(xacc.at[pl.ds(sid * NPS, NPS)],
                    xs_out.at[cid, pl.ds(sid * NPS, NPS)])


def kernel(X, v2e_src, v2e_dst, S_features, W_x, b_x, W_vertex, b_vertex,
           W_group, b_group, W_att, W_e2v, b_e2v):
    x_pad = jnp.pad(X, ((0, NP - N), (0, 0)))
    npad = NNZP - NNZ
    pad_src = (N + jnp.arange(npad, dtype=jnp.int32) % (NP - N)).astype(jnp.int32)
    pad_dst = (M + jnp.arange(npad, dtype=jnp.int32) % (MP - M)).astype(jnp.int32)
    src_all = jnp.concatenate([v2e_src, pad_src])
    dst_all = jnp.concatenate([v2e_dst, pad_dst])
    src_t = src_all.reshape(NW, PC, CHUNK)
    dst_t = dst_all.reshape(NW, PC, CHUNK)
    src_s = jnp.swapaxes(src_all.reshape(NW, CHUNK, PC), 1, 2)
    dst_s = jnp.swapaxes(dst_all.reshape(NW, CHUNK, PC), 1, 2)

    s2 = jnp.pad(S_features, ((0, MP - M), (0, DOUT - DS)))
    w1 = W_e2v[:DOUT]
    w2 = jnp.pad(W_e2v[DOUT:], ((0, DOUT - DS), (0, 0)))
    bx2 = b_x[None, :]
    bv2 = b_vertex[None, :]
    be2 = b_e2v[None, :]

    x_init, g, esv2 = _tc1(x_pad, W_x, W_vertex, W_att, bx2, bv2)
    esv = esv2.reshape(NP)

    num_p, den_p, cnt_p = _sc1(g, esv, src_t.reshape(NW, PC2, CH2),
                               dst_t.reshape(NW, PC2, CH2))
    y = _tc2(num_p, den_p, s2, w1, w2, be2)
    xs_p = _sc2(y, src_s.reshape(NW, PC2, CH2),
                dst_s.reshape(NW, PC2, CH2))
    return _tc3(xs_p, cnt_p, x_init)


# speedup vs baseline: 1.9754x; 1.0128x over previous
import functools

import jax
import jax.numpy as jnp
from jax import lax
from jax.experimental import pallas as pl
from jax.experimental.pallas import tpu as pltpu
from jax.experimental.pallas import tpu_sc as plsc

N = 10000
M = 5000
NNZ = 320000
DIN = 128
DOUT = 64
DS = 10
NEG_SLOPE = 0.2

NC = 2
NS = 16
NW = NC * NS
LANES = 16

NP = 10240
MP = 5120
NPS = NP // NS
MPS = MP // NS

CHUNK = 128
PC = 80
PT = PC * CHUNK
NNZP = NW * PT

CH2 = 256
PC2 = PT // CH2

BLK = 2560
NBN = NP // BLK
BLK2 = 2560
NBM = MP // BLK2


def _elu(x):
    return jnp.where(x > 0, x, jnp.exp(jnp.minimum(x, 0.0)) - 1.0)


def _tc1_body(x_ref, wx_ref, wv_ref, wa_ref, bx_ref, bv_ref,
              xi_ref, g_ref, esv_ref, xi_s, xf_s, sv_s, gmax_s):
    p = pl.program_id(0)
    b = pl.program_id(1)

    @pl.when(p == 0)
    def _():
        x = x_ref[...]
        xf = jnp.dot(x, wv_ref[...], preferred_element_type=jnp.float32) + bv_ref[...]
        xi = jnp.dot(x, wx_ref[...], preferred_element_type=jnp.float32) + bx_ref[...]
        xi_ref[...] = xi
        xi_s[pl.ds(b * BLK, BLK), :] = xi
        xf_s[pl.ds(b * BLK, BLK), :] = xf
        sv = jnp.dot(xf, wa_ref[...], preferred_element_type=jnp.float32)
        sv = jnp.where(sv > 0, sv, NEG_SLOPE * sv)
        sv_s[pl.ds(b * BLK, BLK), :] = sv
        m = jnp.max(sv)

        @pl.when(b == 0)
        def _():
            gmax_s[0] = m

        @pl.when(b > 0)
        def _():
            gmax_s[0] = jnp.maximum(gmax_s[0], m)

    @pl.when(p == 1)
    def _():
        xi_ref[...] = xi_s[pl.ds(b * BLK, BLK), :]
        esv = jnp.exp(sv_s[pl.ds(b * BLK, BLK), :] - gmax_s[0])
        g_ref[...] = xf_s[pl.ds(b * BLK, BLK), :] * esv
        esv_ref[...] = esv


def _tc1(x_pad, w_x, w_vertex, w_att, bx2, bv2):
    return pl.pallas_call(
        _tc1_body,
        grid=(2, NBN),
        in_specs=[
            pl.BlockSpec((BLK, DIN), lambda p, b: ((1 - p) * b, 0)),
            pl.BlockSpec((DIN, DOUT), lambda p, b: (0, 0)),
            pl.BlockSpec((DIN, DOUT), lambda p, b: (0, 0)),
            pl.BlockSpec((DOUT, 1), lambda p, b: (0, 0)),
            pl.BlockSpec((1, DOUT), lambda p, b: (0, 0)),
            pl.BlockSpec((1, DOUT), lambda p, b: (0, 0)),
        ],
        out_specs=[
            pl.BlockSpec((BLK, DOUT), lambda p, b: (b, 0)),
            pl.BlockSpec((BLK, DOUT), lambda p, b: (b, 0)),
            pl.BlockSpec((BLK, 1), lambda p, b: (b, 0)),
        ],
        out_shape=[
            jax.ShapeDtypeStruct((NP, DOUT), jnp.float32),
            jax.ShapeDtypeStruct((NP, DOUT), jnp.float32),
            jax.ShapeDtypeStruct((NP, 1), jnp.float32),
        ],
        scratch_shapes=[
            pltpu.VMEM((NP, DOUT), jnp.float32),
            pltpu.VMEM((NP, DOUT), jnp.float32),
            pltpu.VMEM((NP, 1), jnp.float32),
            pltpu.SMEM((1,), jnp.float32),
        ],
    )(x_pad, w_x, w_vertex, w_att, bx2, bv2)


def _tc2_body(np_ref, dp_ref, s2_ref, w1_ref, w2_ref, be_ref, y_ref):
    num = np_ref[0] + np_ref[1]
    den = jnp.maximum(jnp.sum(dp_ref[...], axis=(0, 1)), 1e-12)
    yv = _elu(num / den[:, None])
    y_ref[...] = (
        jnp.dot(yv, w1_ref[...], preferred_element_type=jnp.float32)
        + jnp.dot(s2_ref[...], w2_ref[...], preferred_element_type=jnp.float32)
        + be_ref[...]
    )


def _tc2(num_p, den_p, s2, w1, w2, be2):
    return pl.pallas_call(
        _tc2_body,
        grid=(NBM,),
        in_specs=[
            pl.BlockSpec((NC, BLK2, DOUT), lambda b: (0, b, 0)),
            pl.BlockSpec((NC, NS, BLK2), lambda b: (0, 0, b)),
            pl.BlockSpec((BLK2, DOUT), lambda b: (b, 0)),
            pl.BlockSpec((DOUT, DOUT), lambda b: (0, 0)),
            pl.BlockSpec((DOUT, DOUT), lambda b: (0, 0)),
            pl.BlockSpec((1, DOUT), lambda b: (0, 0)),
        ],
        out_specs=pl.BlockSpec((BLK2, DOUT), lambda b: (b, 0)),
        out_shape=jax.ShapeDtypeStruct((MP, DOUT), jnp.float32),
    )(num_p, den_p, s2, w1, w2, be2)


def _tc3_body(xp_ref, cp_ref, xi_ref, out_ref):
    xs = xp_ref[0] + xp_ref[1]
    cnt = jnp.maximum(jnp.sum(cp_ref[...], axis=(0, 1)), 1.0)
    out_ref[...] = _elu(xs / cnt[:, None]) + xi_ref[...]


def _tc3(xs_p, cnt_p, x_init):
    return pl.pallas_call(
        _tc3_body,
        grid=(NBN,),
        in_specs=[
            pl.BlockSpec((NC, BLK, DOUT), lambda b: (0, b, 0)),
            pl.BlockSpec((NC, NS, BLK), lambda b: (0, 0, b)),
            pl.BlockSpec((BLK, DOUT), lambda b: (b, 0)),
        ],
        out_specs=pl.BlockSpec((BLK, DOUT), lambda b: (b, 0)),
        out_shape=jax.ShapeDtypeStruct((N, DOUT), jnp.float32),
    )(xs_p, cnt_p, x_init)


_MESH = plsc.VectorSubcoreMesh(core_axis_name="c", subcore_axis_name="s")

_Z16 = functools.partial(jnp.zeros, (LANES,), jnp.float32)


def _zero_1d(ref, n):
    def body(i, _):
        ref[pl.ds(i * LANES, LANES)] = _Z16()
        return 0
    lax.fori_loop(0, n // LANES, body, 0)


def _zero_rows(ref, rows):
    def body(i, _):
        for k in range(DOUT // LANES):
            ref[i, pl.ds(k * LANES, LANES)] = _Z16()
        return 0
    lax.fori_loop(0, rows, body, 0)


@functools.partial(
    pl.kernel,
    out_type=[
        jax.ShapeDtypeStruct((NC, MP, DOUT), jnp.float32),
        jax.ShapeDtypeStruct((NC, NS, MP), jnp.float32),
        jax.ShapeDtypeStruct((NC, NS, NP), jnp.float32),
    ],
    mesh=_MESH,
    scratch_types=[
        pltpu.VMEM((PC2, CH2), jnp.int32),
        pltpu.VMEM((PC2, CH2), jnp.int32),
        pltpu.VMEM((NP,), jnp.float32),
        pltpu.VMEM((2, CH2, DOUT), jnp.float32),
        pltpu.VMEM((MP,), jnp.float32),
        pltpu.VMEM((NP,), jnp.float32),
        pltpu.VMEM_SHARED((MP, DOUT), jnp.float32),
        pltpu.SemaphoreType.DMA,
        pltpu.SemaphoreType.DMA,
        pltpu.SemaphoreType.DMA,
        pltpu.SemaphoreType.DMA,
    ],
    compiler_params=pltpu.CompilerParams(use_tc_tiling_on_sc=False, needs_layout_passes=False),
    name="sc1_v2e",
)
def _sc1(g_hbm, esv_hbm, src_hbm, dst_hbm, num_out, den_out, cnt_out,
         src_v, dst_v, esv_v, rowbuf, den_loc, cnt_loc, num_sh,
         gs0, gs1, ss0, ss1):
    cid = lax.axis_index("c")
    sid = lax.axis_index("s")
    wid = cid * NS + sid

    pltpu.sync_copy(src_hbm.at[wid], src_v)
    pltpu.sync_copy(dst_hbm.at[wid], dst_v)
    pltpu.sync_copy(esv_hbm, esv_v)

    _zero_rows(rowbuf.at[0], CH2)
    _zero_rows(rowbuf.at[1], MPS - CH2)
    _zero_1d(den_loc, MP)
    _zero_1d(cnt_loc, NP)
    pltpu.sync_copy(rowbuf.at[0], num_sh.at[pl.ds(sid * MPS, CH2)])
    pltpu.sync_copy(rowbuf.at[1, pl.ds(0, MPS - CH2)],
                    num_sh.at[pl.ds(sid * MPS + CH2, MPS - CH2)])
    plsc.subcore_barrier()

    ones16 = jnp.ones((LANES,), jnp.float32)
    gsems = (gs0, gs1)
    ssems = (ss0, ss1)
    KR = 2

    pltpu.async_copy(g_hbm.at[src_v.at[0]], rowbuf.at[0], gsems[0])

    def chunk_work(jh, b):
        jj = jh * KR + b
        pb = 1 - b

        def wait_prev_scatter():
            pltpu.make_async_copy(rowbuf.at[pb],
                                  num_sh.at[dst_v.at[0]], ssems[pb]).wait()
        if b == 0:
            @pl.when(jh > 0)
            def _():
                wait_prev_scatter()
        else:
            wait_prev_scatter()

        @pl.when(jj + 1 < PC2)
        def _():
            pltpu.async_copy(g_hbm.at[src_v.at[jj + 1]],
                             rowbuf.at[pb], gsems[pb])

        pltpu.make_async_copy(g_hbm.at[src_v.at[jj]], rowbuf.at[b],
                              gsems[b]).wait()
        pltpu.async_copy(rowbuf.at[b], num_sh.at[dst_v.at[jj]], ssems[b],
                         add=True)
        for k in range(CH2 // LANES):
            sidx = src_v[jj, pl.ds(k * LANES, LANES)]
            didx = dst_v[jj, pl.ds(k * LANES, LANES)]
            e = plsc.load_gather(esv_v, [sidx])
            plsc.addupdate_scatter(den_loc, [didx], e)
            plsc.addupdate_scatter(cnt_loc, [sidx], ones16)

    def body(jh, _):
        for b in range(KR):
            chunk_work(jh, b)
        return 0

    lax.fori_loop(0, PC2 // KR, body, 0)
    pltpu.make_async_copy(rowbuf.at[KR - 1], num_sh.at[dst_v.at[0]],
                          ssems[KR - 1]).wait()

    plsc.subcore_barrier()
    pltpu.sync_copy(num_sh.at[pl.ds(sid * MPS, MPS)],
                    num_out.at[cid, pl.ds(sid * MPS, MPS)])
    pltpu.sync_copy(den_loc, den_out.at[cid, sid])
    pltpu.sync_copy(cnt_loc, cnt_out.at[cid, sid])


@functools.partial(
    pl.kernel,
    out_type=jax.ShapeDtypeStruct((NC, NP, DOUT), jnp.float32),
    mesh=_MESH,
    scratch_types=[
        pltpu.VMEM((PC2, CH2), jnp.int32),
        pltpu.VMEM((PC2, CH2), jnp.int32),
        pltpu.VMEM((4, CH2, DOUT), jnp.float32),
        pltpu.VMEM_SHARED((NP, DOUT), jnp.float32),
        pltpu.SemaphoreType.DMA,
        pltpu.SemaphoreType.DMA,
        pltpu.SemaphoreType.DMA,
        pltpu.SemaphoreType.DMA,
        pltpu.SemaphoreType.DMA,
        pltpu.SemaphoreType.DMA,
        pltpu.SemaphoreType.DMA,
        pltpu.SemaphoreType.DMA,
    ],
    compiler_params=pltpu.CompilerParams(use_tc_tiling_on_sc=False, needs_layout_passes=False),
    name="sc2_e2v",
)
def _sc2(y_hbm, src_hbm, dst_hbm, xs_out,
         src_v, dst_v, rowbuf, xacc,
         gs0, gs1, gs2, gs3, ss0, ss1, ss2, ss3):
    cid = lax.axis_index("c")
    sid = lax.axis_index("s")
    wid = cid * NS + sid

    pltpu.sync_copy(src_hbm.at[wid], src_v)
    pltpu.sync_copy(dst_hbm.at[wid], dst_v)

    for b in range(3):
        _zero_rows(rowbuf.at[b], CH2)
    pltpu.sync_copy(rowbuf.at[0], xacc.at[pl.ds(sid * NPS, CH2)])
    pltpu.sync_copy(rowbuf.at[1], xacc.at[pl.ds(sid * NPS + CH2, CH2)])
    pltpu.sync_copy(rowbuf.at[2, pl.ds(0, NPS - 2 * CH2)],
                    xacc.at[pl.ds(sid * NPS + 2 * CH2, NPS - 2 * CH2)])
    plsc.subcore_barrier()

    gsems = (gs0, gs1, gs2, gs3)
    ssems = (ss0, ss1, ss2, ss3)
    KR = 4

    for b in range(KR - 1):
        pltpu.async_copy(y_hbm.at[dst_v.at[b]], rowbuf.at[b], gsems[b])

    def chunk_work(jh, b):
        jj = jh * KR + b
        pb = (b - 1) % KR

        def wait_prev_scatter():
            pltpu.make_async_copy(rowbuf.at[pb],
                                  xacc.at[src_v.at[0]], ssems[pb]).wait()
        if b == 0:
            @pl.when(jh > 0)
            def _():
                wait_prev_scatter()
        else:
            wait_prev_scatter()

        @pl.when(jj + (KR - 1) < PC2)
        def _():
            pltpu.async_copy(y_hbm.at[dst_v.at[jj + KR - 1]],
                             rowbuf.at[pb], gsems[pb])

        pltpu.make_async_copy(y_hbm.at[dst_v.at[jj]], rowbuf.at[b],
                              gsems[b]).wait()
        pltpu.async_copy(rowbuf.at[b], xacc.at[src_v.at[jj]], ssems[b],
                         add=True)

    def body(jh, _):
        for b in range(KR):
            chunk_work(jh, b)
        return 0

    lax.fori_loop(0, PC2 // KR, body, 0)
    pltpu.make_async_copy(rowbuf.at[KR - 1], xacc.at[src_v.at[0]],
                          ssems[KR - 1]).wait()

    plsc.subcore_barrier()
    pltpu.sync_copy(xacc.at[pl.ds(sid * NPS, NPS)],
                    xs_out.at[cid, pl.ds(sid * NPS, NPS)])


def kernel(X, v2e_src, v2e_dst, S_features, W_x, b_x, W_vertex, b_vertex,
           W_group, b_group, W_att, W_e2v, b_e2v):
    x_pad = jnp.pad(X, ((0, NP - N), (0, 0)))
    npad = NNZP - NNZ
    pad_src = (N + jnp.arange(npad, dtype=jnp.int32) % (NP - N)).astype(jnp.int32)
    pad_dst = (M + jnp.arange(npad, dtype=jnp.int32) % (MP - M)).astype(jnp.int32)
    src_all = jnp.concatenate([v2e_src, pad_src])
    dst_all = jnp.concatenate([v2e_dst, pad_dst])
    src_t = src_all.reshape(NW, PC2, CH2)
    dst_t = dst_all.reshape(NW, PC2, CH2)
    src_s = jnp.swapaxes(src_all.reshape(NW, CHUNK, PC), 1, 2)
    dst_s = jnp.swapaxes(dst_all.reshape(NW, CHUNK, PC), 1, 2)

    s2 = jnp.pad(S_features, ((0, MP - M), (0, DOUT - DS)))
    w1 = W_e2v[:DOUT]
    w2 = jnp.pad(W_e2v[DOUT:], ((0, DOUT - DS), (0, 0)))
    bx2 = b_x[None, :]
    bv2 = b_vertex[None, :]
    be2 = b_e2v[None, :]

    x_init, g, esv2 = _tc1(x_pad, W_x, W_vertex, W_att, bx2, bv2)
    esv = esv2.reshape(NP)

    num_p, den_p, cnt_p = _sc1(g, esv, src_t, dst_t)
    y = _tc2(num_p, den_p, s2, w1, w2, be2)
    xs_p = _sc2(y, src_s.reshape(NW, PC2, CH2),
                dst_s.reshape(NW, PC2, CH2))
    return _tc3(xs_p, cnt_p, x_init)


# speedup vs baseline: 2.0264x; 1.0258x over previous
import functools

import jax
import jax.numpy as jnp
from jax import lax
from jax.experimental import pallas as pl
from jax.experimental.pallas import tpu as pltpu
from jax.experimental.pallas import tpu_sc as plsc

N = 10000
M = 5000
NNZ = 320000
DIN = 128
DOUT = 64
DS = 10
NEG_SLOPE = 0.2

NC = 2
NS = 16
NW = NC * NS
LANES = 16

NP = 10240
MP = 5120
NPS = NP // NS
MPS = MP // NS

CHUNK = 128
PC = 80
PT = PC * CHUNK
NNZP = NW * PT

CH2 = 256
PC2 = PT // CH2

BLK = 2560
NBN = NP // BLK
BLK2 = 2560
NBM = MP // BLK2


def _elu(x):
    return jnp.where(x > 0, x, jnp.exp(jnp.minimum(x, 0.0)) - 1.0)


def _tc1_body(x_ref, wx_ref, wv_ref, wa_ref, bx_ref, bv_ref,
              xi_ref, g_ref, esv_ref, xi_s, xf_s, sv_s, gmax_s):
    p = pl.program_id(0)
    b = pl.program_id(1)

    @pl.when(p == 0)
    def _():
        x = x_ref[...]
        xf = jnp.dot(x, wv_ref[...], preferred_element_type=jnp.float32) + bv_ref[...]
        xi = jnp.dot(x, wx_ref[...], preferred_element_type=jnp.float32) + bx_ref[...]
        xi_ref[...] = xi
        xi_s[pl.ds(b * BLK, BLK), :] = xi
        xf_s[pl.ds(b * BLK, BLK), :] = xf
        sv = jnp.dot(xf, wa_ref[...], preferred_element_type=jnp.float32)
        sv = jnp.where(sv > 0, sv, NEG_SLOPE * sv)
        sv_s[pl.ds(b * BLK, BLK), :] = sv
        rid = lax.broadcasted_iota(jnp.int32, (BLK, 1), 0) + b * BLK
        m = jnp.max(jnp.where(rid < N, sv, -jnp.inf))

        @pl.when(b == 0)
        def _():
            gmax_s[0] = m

        @pl.when(b > 0)
        def _():
            gmax_s[0] = jnp.maximum(gmax_s[0], m)

    @pl.when(p == 1)
    def _():
        xi_ref[...] = xi_s[pl.ds(b * BLK, BLK), :]
        esv = jnp.exp(sv_s[pl.ds(b * BLK, BLK), :] - gmax_s[0])
        g_ref[...] = xf_s[pl.ds(b * BLK, BLK), :] * esv
        esv_ref[...] = esv


def _tc1(x_pad, w_x, w_vertex, w_att, bx2, bv2):
    return pl.pallas_call(
        _tc1_body,
        grid=(2, NBN),
        in_specs=[
            pl.BlockSpec((BLK, DIN), lambda p, b: ((1 - p) * b, 0)),
            pl.BlockSpec((DIN, DOUT), lambda p, b: (0, 0)),
            pl.BlockSpec((DIN, DOUT), lambda p, b: (0, 0)),
            pl.BlockSpec((DOUT, 1), lambda p, b: (0, 0)),
            pl.BlockSpec((1, DOUT), lambda p, b: (0, 0)),
            pl.BlockSpec((1, DOUT), lambda p, b: (0, 0)),
        ],
        out_specs=[
            pl.BlockSpec((BLK, DOUT), lambda p, b: (b, 0)),
            pl.BlockSpec((BLK, DOUT), lambda p, b: (b, 0)),
            pl.BlockSpec((BLK, 1), lambda p, b: (b, 0)),
        ],
        out_shape=[
            jax.ShapeDtypeStruct((NP, DOUT), jnp.float32),
            jax.ShapeDtypeStruct((NP, DOUT), jnp.float32),
            jax.ShapeDtypeStruct((NP, 1), jnp.float32),
        ],
        scratch_shapes=[
            pltpu.VMEM((NP, DOUT), jnp.float32),
            pltpu.VMEM((NP, DOUT), jnp.float32),
            pltpu.VMEM((NP, 1), jnp.float32),
            pltpu.SMEM((1,), jnp.float32),
        ],
    )(x_pad, w_x, w_vertex, w_att, bx2, bv2)


def _tc2_body(np_ref, dp_ref, s2_ref, w1_ref, w2_ref, be_ref, y_ref):
    num = np_ref[0] + np_ref[1]
    den = jnp.maximum(jnp.sum(dp_ref[...], axis=(0, 1)), 1e-12)
    yv = _elu(num / den[:, None])
    y_ref[...] = (
        jnp.dot(yv, w1_ref[...], preferred_element_type=jnp.float32)
        + jnp.dot(s2_ref[...], w2_ref[...], preferred_element_type=jnp.float32)
        + be_ref[...]
    )


def _tc2(num_p, den_p, s2, w1, w2, be2):
    return pl.pallas_call(
        _tc2_body,
        grid=(NBM,),
        in_specs=[
            pl.BlockSpec((NC, BLK2, DOUT), lambda b: (0, b, 0)),
            pl.BlockSpec((NC, NS, BLK2), lambda b: (0, 0, b)),
            pl.BlockSpec((BLK2, DOUT), lambda b: (b, 0)),
            pl.BlockSpec((DOUT, DOUT), lambda b: (0, 0)),
            pl.BlockSpec((DOUT, DOUT), lambda b: (0, 0)),
            pl.BlockSpec((1, DOUT), lambda b: (0, 0)),
        ],
        out_specs=pl.BlockSpec((BLK2, DOUT), lambda b: (b, 0)),
        out_shape=jax.ShapeDtypeStruct((MP, DOUT), jnp.float32),
    )(num_p, den_p, s2, w1, w2, be2)


def _tc3_body(xp_ref, cp_ref, xi_ref, out_ref):
    xs = xp_ref[0] + xp_ref[1]
    cnt = jnp.maximum(jnp.sum(cp_ref[...], axis=(0, 1)), 1.0)
    out_ref[...] = _elu(xs / cnt[:, None]) + xi_ref[...]


def _tc3(xs_p, cnt_p, x_init):
    return pl.pallas_call(
        _tc3_body,
        grid=(NBN,),
        in_specs=[
            pl.BlockSpec((NC, BLK, DOUT), lambda b: (0, b, 0)),
            pl.BlockSpec((NC, NS, BLK), lambda b: (0, 0, b)),
            pl.BlockSpec((BLK, DOUT), lambda b: (b, 0)),
        ],
        out_specs=pl.BlockSpec((BLK, DOUT), lambda b: (b, 0)),
        out_shape=jax.ShapeDtypeStruct((N, DOUT), jnp.float32),
    )(xs_p, cnt_p, x_init)


_MESH = plsc.VectorSubcoreMesh(core_axis_name="c", subcore_axis_name="s")

_Z16 = functools.partial(jnp.zeros, (LANES,), jnp.float32)


def _zero_1d(ref, n):
    def body(i, _):
        ref[pl.ds(i * LANES, LANES)] = _Z16()
        return 0
    lax.fori_loop(0, n // LANES, body, 0)


def _zero_rows(ref, rows):
    def body(i, _):
        for k in range(DOUT // LANES):
            ref[i, pl.ds(k * LANES, LANES)] = _Z16()
        return 0
    lax.fori_loop(0, rows, body, 0)


@functools.partial(
    pl.kernel,
    out_type=[
        jax.ShapeDtypeStruct((NC, MP, DOUT), jnp.float32),
        jax.ShapeDtypeStruct((NC, NS, MP), jnp.float32),
        jax.ShapeDtypeStruct((NC, NS, NP), jnp.float32),
    ],
    mesh=_MESH,
    scratch_types=[
        pltpu.VMEM((PC2, CH2), jnp.int32),
        pltpu.VMEM((PC2, CH2), jnp.int32),
        pltpu.VMEM((NP,), jnp.float32),
        pltpu.VMEM((2, CH2, DOUT), jnp.float32),
        pltpu.VMEM((MP,), jnp.float32),
        pltpu.VMEM((NP,), jnp.float32),
        pltpu.VMEM_SHARED((MP, DOUT), jnp.float32),
        pltpu.SemaphoreType.DMA,
        pltpu.SemaphoreType.DMA,
        pltpu.SemaphoreType.DMA,
        pltpu.SemaphoreType.DMA,
    ],
    compiler_params=pltpu.CompilerParams(use_tc_tiling_on_sc=False, needs_layout_passes=False),
    name="sc1_v2e",
)
def _sc1(g_hbm, esv_hbm, src_hbm, dst_hbm, num_out, den_out, cnt_out,
         src_v, dst_v, esv_v, rowbuf, den_loc, cnt_loc, num_sh,
         gs0, gs1, ss0, ss1):
    cid = lax.axis_index("c")
    sid = lax.axis_index("s")
    wid = cid * NS + sid

    pltpu.sync_copy(src_hbm.at[wid], src_v)
    pltpu.sync_copy(dst_hbm.at[wid], dst_v)
    pltpu.sync_copy(esv_hbm, esv_v)

    _zero_rows(rowbuf.at[0], CH2)
    _zero_rows(rowbuf.at[1], MPS - CH2)
    _zero_1d(den_loc, MP)
    _zero_1d(cnt_loc, NP)
    pltpu.sync_copy(rowbuf.at[0], num_sh.at[pl.ds(sid * MPS, CH2)])
    pltpu.sync_copy(rowbuf.at[1, pl.ds(0, MPS - CH2)],
                    num_sh.at[pl.ds(sid * MPS + CH2, MPS - CH2)])
    plsc.subcore_barrier()

    ones16 = jnp.ones((LANES,), jnp.float32)
    gsems = (gs0, gs1)
    ssems = (ss0, ss1)
    KR = 2

    pltpu.async_copy(g_hbm.at[src_v.at[0]], rowbuf.at[0], gsems[0])

    def chunk_work(jh, b):
        jj = jh * KR + b
        pb = 1 - b

        def wait_prev_scatter():
            pltpu.make_async_copy(rowbuf.at[pb],
                                  num_sh.at[dst_v.at[0]], ssems[pb]).wait()
        if b == 0:
            @pl.when(jh > 0)
            def _():
                wait_prev_scatter()
        else:
            wait_prev_scatter()

        @pl.when(jj + 1 < PC2)
        def _():
            pltpu.async_copy(g_hbm.at[src_v.at[jj + 1]],
                             rowbuf.at[pb], gsems[pb])

        pltpu.make_async_copy(g_hbm.at[src_v.at[jj]], rowbuf.at[b],
                              gsems[b]).wait()
        pltpu.async_copy(rowbuf.at[b], num_sh.at[dst_v.at[jj]], ssems[b],
                         add=True)
        for k in range(CH2 // LANES):
            sidx = src_v[jj, pl.ds(k * LANES, LANES)]
            didx = dst_v[jj, pl.ds(k * LANES, LANES)]
            e = plsc.load_gather(esv_v, [sidx])
            plsc.addupdate_scatter(den_loc, [didx], e)
            plsc.addupdate_scatter(cnt_loc, [sidx], ones16)

    def body(jh, _):
        for b in range(KR):
            chunk_work(jh, b)
        return 0

    lax.fori_loop(0, PC2 // KR, body, 0)
    pltpu.make_async_copy(rowbuf.at[KR - 1], num_sh.at[dst_v.at[0]],
                          ssems[KR - 1]).wait()

    plsc.subcore_barrier()
    pltpu.sync_copy(num_sh.at[pl.ds(sid * MPS, MPS)],
                    num_out.at[cid, pl.ds(sid * MPS, MPS)])
    pltpu.sync_copy(den_loc, den_out.at[cid, sid])
    pltpu.sync_copy(cnt_loc, cnt_out.at[cid, sid])


@functools.partial(
    pl.kernel,
    out_type=jax.ShapeDtypeStruct((NC, NP, DOUT), jnp.float32),
    mesh=_MESH,
    scratch_types=[
        pltpu.VMEM((PC2, CH2), jnp.int32),
        pltpu.VMEM((PC2, CH2), jnp.int32),
        pltpu.VMEM((4, CH2, DOUT), jnp.float32),
        pltpu.VMEM_SHARED((NP, DOUT), jnp.float32),
        pltpu.SemaphoreType.DMA,
        pltpu.SemaphoreType.DMA,
        pltpu.SemaphoreType.DMA,
        pltpu.SemaphoreType.DMA,
        pltpu.SemaphoreType.DMA,
        pltpu.SemaphoreType.DMA,
        pltpu.SemaphoreType.DMA,
        pltpu.SemaphoreType.DMA,
    ],
    compiler_params=pltpu.CompilerParams(use_tc_tiling_on_sc=False, needs_layout_passes=False),
    name="sc2_e2v",
)
def _sc2(y_hbm, src_hbm, dst_hbm, xs_out,
         src_v, dst_v, rowbuf, xacc,
         gs0, gs1, gs2, gs3, ss0, ss1, ss2, ss3):
    cid = lax.axis_index("c")
    sid = lax.axis_index("s")
    wid = cid * NS + sid

    pltpu.sync_copy(src_hbm.at[wid], src_v)
    pltpu.sync_copy(dst_hbm.at[wid], dst_v)

    for b in range(3):
        _zero_rows(rowbuf.at[b], CH2)
    pltpu.sync_copy(rowbuf.at[0], xacc.at[pl.ds(sid * NPS, CH2)])
    pltpu.sync_copy(rowbuf.at[1], xacc.at[pl.ds(sid * NPS + CH2, CH2)])
    pltpu.sync_copy(rowbuf.at[2, pl.ds(0, NPS - 2 * CH2)],
                    xacc.at[pl.ds(sid * NPS + 2 * CH2, NPS - 2 * CH2)])
    plsc.subcore_barrier()

    gsems = (gs0, gs1, gs2, gs3)
    ssems = (ss0, ss1, ss2, ss3)
    KR = 4

    for b in range(KR - 1):
        pltpu.async_copy(y_hbm.at[dst_v.at[b]], rowbuf.at[b], gsems[b])

    def chunk_work(jh, b):
        jj = jh * KR + b
        pb = (b - 1) % KR

        def wait_prev_scatter():
            pltpu.make_async_copy(rowbuf.at[pb],
                                  xacc.at[src_v.at[0]], ssems[pb]).wait()
        if b == 0:
            @pl.when(jh > 0)
            def _():
                wait_prev_scatter()
        else:
            wait_prev_scatter()

        @pl.when(jj + (KR - 1) < PC2)
        def _():
            pltpu.async_copy(y_hbm.at[dst_v.at[jj + KR - 1]],
                             rowbuf.at[pb], gsems[pb])

        pltpu.make_async_copy(y_hbm.at[dst_v.at[jj]], rowbuf.at[b],
                              gsems[b]).wait()
        pltpu.async_copy(rowbuf.at[b], xacc.at[src_v.at[jj]], ssems[b],
                         add=True)

    def body(jh, _):
        for b in range(KR):
            chunk_work(jh, b)
        return 0

    lax.fori_loop(0, PC2 // KR, body, 0)
    pltpu.make_async_copy(rowbuf.at[KR - 1], xacc.at[src_v.at[0]],
                          ssems[KR - 1]).wait()

    plsc.subcore_barrier()
    pltpu.sync_copy(xacc.at[pl.ds(sid * NPS, NPS)],
                    xs_out.at[cid, pl.ds(sid * NPS, NPS)])


def kernel(X, v2e_src, v2e_dst, S_features, W_x, b_x, W_vertex, b_vertex,
           W_group, b_group, W_att, W_e2v, b_e2v):
    npad = NNZP - NNZ
    pad_src = (N + jnp.arange(npad, dtype=jnp.int32) % (NP - N)).astype(jnp.int32)
    pad_dst = (M + jnp.arange(npad, dtype=jnp.int32) % (MP - M)).astype(jnp.int32)
    src_all = jnp.concatenate([v2e_src, pad_src])
    dst_all = jnp.concatenate([v2e_dst, pad_dst])
    src_t = src_all.reshape(NW, PC2, CH2)
    dst_t = dst_all.reshape(NW, PC2, CH2)
    src_s = jnp.swapaxes(src_all.reshape(NW, CHUNK, PC), 1, 2)
    dst_s = jnp.swapaxes(dst_all.reshape(NW, CHUNK, PC), 1, 2)

    s2 = jnp.pad(S_features, ((0, MP - M), (0, DOUT - DS)))
    w1 = W_e2v[:DOUT]
    w2 = jnp.pad(W_e2v[DOUT:], ((0, DOUT - DS), (0, 0)))
    bx2 = b_x[None, :]
    bv2 = b_vertex[None, :]
    be2 = b_e2v[None, :]

    x_init, g, esv2 = _tc1(X, W_x, W_vertex, W_att, bx2, bv2)
    esv = esv2.reshape(NP)

    num_p, den_p, cnt_p = _sc1(g, esv, src_t, dst_t)
    y = _tc2(num_p, den_p, s2, w1, w2, be2)
    xs_p = _sc2(y, src_s.reshape(NW, PC2, CH2),
                dst_s.reshape(NW, PC2, CH2))
    return _tc3(xs_p, cnt_p, x_init)
